# Initial kernel scaffold; baseline (speedup 1.0000x reference)
import jax
import jax.numpy as jnp
from jax.experimental import pallas as pl


def kernel(x, edge_index, pre_W1, pre_b1, pre_W2, pre_b2, conv_W, conv_b, pol_W1, pol_b1, pol_W2, pol_b2, val_W1, val_b1, val_W2, val_b2):
    raise NotImplementedError("write your pallas kernel here")


# speedup vs baseline: 34.6257x; 34.6257x over previous
import functools

import jax
import jax.numpy as jnp
from jax import lax
from jax.experimental import pallas as pl
from jax.experimental.pallas import tpu as pltpu
from jax.experimental.pallas import tpu_sc as plsc

N = 10000
F = 16
NC, NS, LANES = 2, 16, 16
NW = NC * NS
BS = 128
NBW = 80
EPAD = NW * NBW * BS
NPAD = 10240
RPT = NPAD // NS

_HI = lax.Precision.HIGHEST

_mesh = plsc.VectorSubcoreMesh(
    core_axis_name="c", subcore_axis_name="s", num_cores=NC, num_subcores=NS
)

_sc_params = pltpu.CompilerParams(use_tc_tiling_on_sc=False)


@functools.partial(
    pl.kernel,
    out_type=jax.ShapeDtypeStruct((NC, NPAD), jnp.float32),
    mesh=_mesh,
    scratch_types=[
        pltpu.VMEM((NBW, BS), jnp.int32),
        pltpu.VMEM((BS,), jnp.float32),
        pltpu.VMEM((RPT,), jnp.float32),
        pltpu.VMEM_SHARED((NPAD,), jnp.float32),
    ],
    compiler_params=_sc_params,
)
def _deg_kernel(dst_hbm, degp_hbm, dstv, onesv, zb, acc):
    c = lax.axis_index("c")
    s = lax.axis_index("s")
    wid = c * NS + s

    def _zero(i, carry):
        zb[pl.ds(i * LANES, LANES)] = jnp.zeros((LANES,), jnp.float32)
        return carry

    lax.fori_loop(0, RPT // LANES, _zero, 0)

    def _one(i, carry):
        onesv[pl.ds(i * LANES, LANES)] = jnp.ones((LANES,), jnp.float32)
        return carry

    lax.fori_loop(0, BS // LANES, _one, 0)

    pltpu.sync_copy(zb, acc.at[pl.ds(s * RPT, RPT)])
    pltpu.sync_copy(dst_hbm.at[wid], dstv)
    plsc.subcore_barrier()

    def _scat(j, carry):
        pltpu.sync_copy(onesv, acc.at[dstv.at[j]], add=True)
        return carry

    lax.fori_loop(0, NBW, _scat, 0)
    plsc.subcore_barrier()
    pltpu.sync_copy(acc.at[pl.ds(s * RPT, RPT)],
                    degp_hbm.at[c, pl.ds(s * RPT, RPT)])


@functools.partial(
    pl.kernel,
    out_type=jax.ShapeDtypeStruct((NC, NPAD, F), jnp.float32),
    mesh=_mesh,
    scratch_types=[
        pltpu.VMEM((NBW, BS), jnp.int32),
        pltpu.VMEM((NBW, BS), jnp.int32),
        pltpu.VMEM((BS, F), jnp.float32),
        pltpu.VMEM((BS, F), jnp.float32),
        pltpu.VMEM((RPT, F), jnp.float32),
        pltpu.VMEM_SHARED((NPAD, F), jnp.float32),
        pltpu.SemaphoreType.DMA,
        pltpu.SemaphoreType.DMA,
    ],
    compiler_params=_sc_params,
)
def _scatter_kernel(src_hbm, dst_hbm, g_hbm, out_hbm,
                    srcv, dstv, rows0, rows1, zb, acc, sem0, sem1):
    c = lax.axis_index("c")
    s = lax.axis_index("s")
    wid = c * NS + s

    def _zero(i, carry):
        zb[i, :] = jnp.zeros((LANES,), jnp.float32)
        return carry

    lax.fori_loop(0, RPT, _zero, 0)
    pltpu.sync_copy(zb, acc.at[pl.ds(s * RPT, RPT)])
    pltpu.sync_copy(src_hbm.at[wid], srcv)
    pltpu.sync_copy(dst_hbm.at[wid], dstv)
    plsc.subcore_barrier()

    pltpu.async_copy(g_hbm.at[srcv.at[0]], rows0, sem0)

    def _body(jo, carry):
        j0 = 2 * jo
        pltpu.async_copy(g_hbm.at[srcv.at[j0 + 1]], rows1, sem1)
        pltpu.make_async_copy(g_hbm.at[srcv.at[j0]], rows0, sem0).wait()
        pltpu.sync_copy(rows0, acc.at[dstv.at[j0]], add=True)

        @pl.when(jo < NBW // 2 - 1)
        def _():
            pltpu.async_copy(g_hbm.at[srcv.at[j0 + 2]], rows0, sem0)

        pltpu.make_async_copy(g_hbm.at[srcv.at[j0 + 1]], rows1, sem1).wait()
        pltpu.sync_copy(rows1, acc.at[dstv.at[j0 + 1]], add=True)
        return carry

    lax.fori_loop(0, NBW // 2, _body, 0)
    plsc.subcore_barrier()
    pltpu.sync_copy(acc.at[pl.ds(s * RPT, RPT)],
                    out_hbm.at[c, pl.ds(s * RPT, RPT)])


BT = 2000
GRID = N // BT


def _pre_body(x_ref, degt_ref, w1_ref, b1_ref, w2_ref, b2_ref, cw0_ref,
              g1_ref, dinv_ref):
    xb = x_ref[...]
    h = jnp.maximum(jnp.dot(xb, w1_ref[...], precision=_HI) + b1_ref[...], 0.0)
    h = jnp.dot(h, w2_ref[...], precision=_HI) + b2_ref[...]
    d = degt_ref[...]
    deg = d[:, 0:1] + d[:, 1:2] + 1.0
    dinv = lax.rsqrt(deg)
    g1_ref[...] = jnp.dot(h, cw0_ref[...], precision=_HI) * dinv
    dinv_ref[...] = dinv


def _pre_call(x, degt, w1, b1, w2, b2, cw0):
    return pl.pallas_call(
        _pre_body,
        grid=(GRID,),
        in_specs=[
            pl.BlockSpec((BT, 128), lambda i: (i, 0)),
            pl.BlockSpec((BT, 2), lambda i: (i, 0)),
            pl.BlockSpec((128, 32), lambda i: (0, 0)),
            pl.BlockSpec((1, 32), lambda i: (0, 0)),
            pl.BlockSpec((32, 64), lambda i: (0, 0)),
            pl.BlockSpec((1, 64), lambda i: (0, 0)),
            pl.BlockSpec((64, F), lambda i: (0, 0)),
        ],
        out_specs=[
            pl.BlockSpec((BT, F), lambda i: (i, 0)),
            pl.BlockSpec((BT, 1), lambda i: (i, 0)),
        ],
        out_shape=[
            jax.ShapeDtypeStruct((N, F), jnp.float32),
            jax.ShapeDtypeStruct((N, 1), jnp.float32),
        ],
    )(x, degt, w1, b1, w2, b2, cw0)


def _mid_body(s0_ref, s1_ref, g_ref, dinv_ref, b_ref, wn_ref, gout_ref):
    dinv = dinv_ref[...]
    v = (s0_ref[...] + s1_ref[...] + g_ref[...]) * dinv + b_ref[...]
    h = jnp.where(v > 0, v, jnp.exp(v) - 1.0)
    gout_ref[...] = jnp.dot(h, wn_ref[...], precision=_HI) * dinv


def _mid_call(s0, s1, g, dinv, b, wn):
    return pl.pallas_call(
        _mid_body,
        grid=(GRID,),
        in_specs=[
            pl.BlockSpec((BT, F), lambda i: (i, 0)),
            pl.BlockSpec((BT, F), lambda i: (i, 0)),
            pl.BlockSpec((BT, F), lambda i: (i, 0)),
            pl.BlockSpec((BT, 1), lambda i: (i, 0)),
            pl.BlockSpec((1, F), lambda i: (0, 0)),
            pl.BlockSpec((F, F), lambda i: (0, 0)),
        ],
        out_specs=pl.BlockSpec((BT, F), lambda i: (i, 0)),
        out_shape=jax.ShapeDtypeStruct((N, F), jnp.float32),
    )(s0, s1, g, dinv, b, wn)


def _fin_body(s0_ref, s1_ref, g_ref, dinv_ref, b4_ref,
              pw1_ref, pb1_ref, pw2_ref, pb2_ref,
              vw1_ref, vb1_ref, vw2_ref, vb2_ref,
              proba_ref, value_ref, accum):
    i = pl.program_id(0)
    dinv = dinv_ref[...]
    v = (s0_ref[...] + s1_ref[...] + g_ref[...]) * dinv + b4_ref[...]
    h = jnp.where(v > 0, v, jnp.exp(v) - 1.0)
    p = jnp.dot(h, pw1_ref[...], precision=_HI) + pb1_ref[...]
    p = jnp.where(p > 0, p, jnp.exp(p) - 1.0)
    proba_ref[...] = jnp.dot(p, pw2_ref[...], precision=_HI) + pb2_ref[...]

    bs = jnp.sum(h, axis=0, keepdims=True)

    @pl.when(i == 0)
    def _():
        accum[...] = bs

    @pl.when(i > 0)
    def _():
        accum[...] = accum[...] + bs

    @pl.when(i == GRID - 1)
    def _():
        m = accum[...] * (1.0 / N)
        vv = jnp.dot(m, vw1_ref[...], precision=_HI) + vb1_ref[...]
        vv = jnp.where(vv > 0, vv, jnp.exp(vv) - 1.0)
        value_ref[...] = jnp.dot(vv, vw2_ref[...], precision=_HI) + vb2_ref[...]


def _fin_call(s0, s1, g, dinv, b4, pw1, pb1, pw2, pb2, vw1, vb1, vw2, vb2):
    return pl.pallas_call(
        _fin_body,
        grid=(GRID,),
        in_specs=[
            pl.BlockSpec((BT, F), lambda i: (i, 0)),
            pl.BlockSpec((BT, F), lambda i: (i, 0)),
            pl.BlockSpec((BT, F), lambda i: (i, 0)),
            pl.BlockSpec((BT, 1), lambda i: (i, 0)),
            pl.BlockSpec((1, F), lambda i: (0, 0)),
            pl.BlockSpec((F, F), lambda i: (0, 0)),
            pl.BlockSpec((1, F), lambda i: (0, 0)),
            pl.BlockSpec((F, 1), lambda i: (0, 0)),
            pl.BlockSpec((1, 1), lambda i: (0, 0)),
            pl.BlockSpec((F, F), lambda i: (0, 0)),
            pl.BlockSpec((1, F), lambda i: (0, 0)),
            pl.BlockSpec((F, 1), lambda i: (0, 0)),
            pl.BlockSpec((1, 1), lambda i: (0, 0)),
        ],
        out_specs=[
            pl.BlockSpec((BT, 1), lambda i: (i, 0)),
            pl.BlockSpec((1, 1), lambda i: (0, 0)),
        ],
        out_shape=[
            jax.ShapeDtypeStruct((N, 1), jnp.float32),
            jax.ShapeDtypeStruct((1, 1), jnp.float32),
        ],
        scratch_shapes=[pltpu.VMEM((1, F), jnp.float32)],
    )(s0, s1, g, dinv, b4, pw1, pb1, pw2, pb2, vw1, vb1, vw2, vb2)


def kernel(x, edge_index, pre_W1, pre_b1, pre_W2, pre_b2, conv_W, conv_b,
           pol_W1, pol_b1, pol_W2, pol_b2, val_W1, val_b1, val_W2, val_b2):
    e = edge_index.shape[1]
    src = edge_index[0].astype(jnp.int32)
    dst = edge_index[1].astype(jnp.int32)
    npad = EPAD - e
    ar = jnp.arange(npad, dtype=jnp.int32)
    srcp = jnp.concatenate([src, ar % 256]).reshape(NW, NBW, BS)
    dstp = jnp.concatenate([dst, N + ar % (NPAD - N)]).reshape(NW, NBW, BS)

    degp = _deg_kernel(dstp)
    degt = degp[:, :N].T
    g, dinv = _pre_call(x, degt, pre_W1, pre_b1.reshape(1, -1),
                        pre_W2, pre_b2.reshape(1, -1), conv_W[0])
    for i in range(3):
        spp = _scatter_kernel(srcp, dstp, g)
        g = _mid_call(spp[0, :N], spp[1, :N], g, dinv,
                      conv_b[i].reshape(1, -1), conv_W[i + 1])
    spp = _scatter_kernel(srcp, dstp, g)
    proba, value = _fin_call(
        spp[0, :N], spp[1, :N], g, dinv, conv_b[3].reshape(1, -1),
        pol_W1, pol_b1.reshape(1, -1), pol_W2, pol_b2.reshape(1, 1),
        val_W1, val_b1.reshape(1, -1), val_W2, val_b2.reshape(1, 1))
    return (proba, value)


# speedup vs baseline: 36.2959x; 1.0482x over previous
import functools

import jax
import jax.numpy as jnp
from jax import lax
from jax.experimental import pallas as pl
from jax.experimental.pallas import tpu as pltpu
from jax.experimental.pallas import tpu_sc as plsc

N = 10000
F = 16
NC, NS, LANES = 2, 16, 16
NW = NC * NS
BS = 128
NBW = 80
EPAD = NW * NBW * BS
NPAD = 10240
RPT = NPAD // NS

_HI = lax.Precision.HIGHEST

_mesh = plsc.VectorSubcoreMesh(
    core_axis_name="c", subcore_axis_name="s", num_cores=NC, num_subcores=NS
)

_sc_params = pltpu.CompilerParams(use_tc_tiling_on_sc=False)


@functools.partial(
    pl.kernel,
    out_type=jax.ShapeDtypeStruct((NC, NPAD), jnp.float32),
    mesh=_mesh,
    scratch_types=[
        pltpu.VMEM((NBW, BS), jnp.int32),
        pltpu.VMEM((BS,), jnp.float32),
        pltpu.VMEM((RPT,), jnp.float32),
        pltpu.VMEM_SHARED((NPAD,), jnp.float32),
    ],
    compiler_params=_sc_params,
)
def _deg_kernel(dst_hbm, degp_hbm, dstv, onesv, zb, acc):
    c = lax.axis_index("c")
    s = lax.axis_index("s")
    wid = c * NS + s

    def _zero(i, carry):
        zb[pl.ds(i * LANES, LANES)] = jnp.zeros((LANES,), jnp.float32)
        return carry

    lax.fori_loop(0, RPT // LANES, _zero, 0)

    def _one(i, carry):
        onesv[pl.ds(i * LANES, LANES)] = jnp.ones((LANES,), jnp.float32)
        return carry

    lax.fori_loop(0, BS // LANES, _one, 0)

    pltpu.sync_copy(zb, acc.at[pl.ds(s * RPT, RPT)])
    pltpu.sync_copy(dst_hbm.at[wid], dstv)
    plsc.subcore_barrier()

    def _scat(j, carry):
        pltpu.sync_copy(onesv, acc.at[dstv.at[j]], add=True)
        return carry

    lax.fori_loop(0, NBW, _scat, 0)
    plsc.subcore_barrier()
    pltpu.sync_copy(acc.at[pl.ds(s * RPT, RPT)],
                    degp_hbm.at[c, pl.ds(s * RPT, RPT)])


@functools.partial(
    pl.kernel,
    out_type=jax.ShapeDtypeStruct((NC, NPAD, F), jnp.float32),
    mesh=_mesh,
    scratch_types=[
        pltpu.VMEM((NBW, BS), jnp.int32),
        pltpu.VMEM((NBW, BS), jnp.int32),
        pltpu.VMEM((4, BS, F), jnp.float32),
        pltpu.VMEM((RPT, F), jnp.float32),
        pltpu.VMEM_SHARED((NPAD, F), jnp.float32),
        [pltpu.SemaphoreType.DMA] * 4,
        [pltpu.SemaphoreType.DMA] * 4,
    ],
    compiler_params=_sc_params,
)
def _scatter_kernel(src_hbm, dst_hbm, g_hbm, out_hbm,
                    srcv, dstv, rows, zb, acc, gsems, ssems):
    c = lax.axis_index("c")
    s = lax.axis_index("s")
    wid = c * NS + s

    def _zero(i, carry):
        zb[i, :] = jnp.zeros((LANES,), jnp.float32)
        return carry

    lax.fori_loop(0, RPT, _zero, 0)
    pltpu.sync_copy(zb, acc.at[pl.ds(s * RPT, RPT)])
    pltpu.sync_copy(src_hbm.at[wid], srcv)
    pltpu.sync_copy(dst_hbm.at[wid], dstv)
    plsc.subcore_barrier()

    def _fire_gather(j, b):
        pltpu.async_copy(g_hbm.at[srcv.at[j]], rows.at[b], gsems[b])

    def _fire_scatter(j, b):
        pltpu.async_copy(rows.at[b], acc.at[dstv.at[j]], ssems[b], add=True)

    def _drain_gather(j, b):
        pltpu.make_async_copy(g_hbm.at[srcv.at[j]], rows.at[b], gsems[b]).wait()

    def _drain_scatter(j, b):
        pltpu.make_async_copy(rows.at[b], acc.at[dstv.at[j]], ssems[b]).wait()

    _fire_gather(0, 0)
    _fire_gather(1, 1)

    def _body(jo, carry):
        j0 = 4 * jo
        for u in range(4):
            j = j0 + u
            _drain_gather(j, u)
            _fire_scatter(j, u)

            @pl.when(j >= 2)
            def _():
                _drain_scatter(j - 2, (u - 2) % 4)

            @pl.when(j + 2 < NBW)
            def _():
                _fire_gather(j + 2, (u + 2) % 4)
        return carry

    lax.fori_loop(0, NBW // 4, _body, 0)
    _drain_scatter(NBW - 2, 2)
    _drain_scatter(NBW - 1, 3)
    plsc.subcore_barrier()
    pltpu.sync_copy(acc.at[pl.ds(s * RPT, RPT)],
                    out_hbm.at[c, pl.ds(s * RPT, RPT)])


BT = 2000
GRID = N // BT


def _pre_body(x_ref, degt_ref, w1_ref, b1_ref, w2_ref, b2_ref, cw0_ref,
              g1_ref, dinv_ref):
    xb = x_ref[...]
    h = jnp.maximum(jnp.dot(xb, w1_ref[...], precision=_HI) + b1_ref[...], 0.0)
    h = jnp.dot(h, w2_ref[...], precision=_HI) + b2_ref[...]
    d = degt_ref[...]
    deg = d[:, 0:1] + d[:, 1:2] + 1.0
    dinv = lax.rsqrt(deg)
    g1_ref[...] = jnp.dot(h, cw0_ref[...], precision=_HI) * dinv
    dinv_ref[...] = dinv


def _pre_call(x, degt, w1, b1, w2, b2, cw0):
    return pl.pallas_call(
        _pre_body,
        grid=(GRID,),
        in_specs=[
            pl.BlockSpec((BT, 128), lambda i: (i, 0)),
            pl.BlockSpec((BT, 2), lambda i: (i, 0)),
            pl.BlockSpec((128, 32), lambda i: (0, 0)),
            pl.BlockSpec((1, 32), lambda i: (0, 0)),
            pl.BlockSpec((32, 64), lambda i: (0, 0)),
            pl.BlockSpec((1, 64), lambda i: (0, 0)),
            pl.BlockSpec((64, F), lambda i: (0, 0)),
        ],
        out_specs=[
            pl.BlockSpec((BT, F), lambda i: (i, 0)),
            pl.BlockSpec((BT, 1), lambda i: (i, 0)),
        ],
        out_shape=[
            jax.ShapeDtypeStruct((N, F), jnp.float32),
            jax.ShapeDtypeStruct((N, 1), jnp.float32),
        ],
    )(x, degt, w1, b1, w2, b2, cw0)


def _mid_body(s0_ref, s1_ref, g_ref, dinv_ref, b_ref, wn_ref, gout_ref):
    dinv = dinv_ref[...]
    v = (s0_ref[...] + s1_ref[...] + g_ref[...]) * dinv + b_ref[...]
    h = jnp.where(v > 0, v, jnp.exp(v) - 1.0)
    gout_ref[...] = jnp.dot(h, wn_ref[...], precision=_HI) * dinv


def _mid_call(s0, s1, g, dinv, b, wn):
    return pl.pallas_call(
        _mid_body,
        grid=(GRID,),
        in_specs=[
            pl.BlockSpec((BT, F), lambda i: (i, 0)),
            pl.BlockSpec((BT, F), lambda i: (i, 0)),
            pl.BlockSpec((BT, F), lambda i: (i, 0)),
            pl.BlockSpec((BT, 1), lambda i: (i, 0)),
            pl.BlockSpec((1, F), lambda i: (0, 0)),
            pl.BlockSpec((F, F), lambda i: (0, 0)),
        ],
        out_specs=pl.BlockSpec((BT, F), lambda i: (i, 0)),
        out_shape=jax.ShapeDtypeStruct((N, F), jnp.float32),
    )(s0, s1, g, dinv, b, wn)


def _fin_body(s0_ref, s1_ref, g_ref, dinv_ref, b4_ref,
              pw1_ref, pb1_ref, pw2_ref, pb2_ref,
              vw1_ref, vb1_ref, vw2_ref, vb2_ref,
              proba_ref, value_ref, accum):
    i = pl.program_id(0)
    dinv = dinv_ref[...]
    v = (s0_ref[...] + s1_ref[...] + g_ref[...]) * dinv + b4_ref[...]
    h = jnp.where(v > 0, v, jnp.exp(v) - 1.0)
    p = jnp.dot(h, pw1_ref[...], precision=_HI) + pb1_ref[...]
    p = jnp.where(p > 0, p, jnp.exp(p) - 1.0)
    proba_ref[...] = jnp.dot(p, pw2_ref[...], precision=_HI) + pb2_ref[...]

    bs = jnp.sum(h, axis=0, keepdims=True)

    @pl.when(i == 0)
    def _():
        accum[...] = bs

    @pl.when(i > 0)
    def _():
        accum[...] = accum[...] + bs

    @pl.when(i == GRID - 1)
    def _():
        m = accum[...] * (1.0 / N)
        vv = jnp.dot(m, vw1_ref[...], precision=_HI) + vb1_ref[...]
        vv = jnp.where(vv > 0, vv, jnp.exp(vv) - 1.0)
        value_ref[...] = jnp.dot(vv, vw2_ref[...], precision=_HI) + vb2_ref[...]


def _fin_call(s0, s1, g, dinv, b4, pw1, pb1, pw2, pb2, vw1, vb1, vw2, vb2):
    return pl.pallas_call(
        _fin_body,
        grid=(GRID,),
        in_specs=[
            pl.BlockSpec((BT, F), lambda i: (i, 0)),
            pl.BlockSpec((BT, F), lambda i: (i, 0)),
            pl.BlockSpec((BT, F), lambda i: (i, 0)),
            pl.BlockSpec((BT, 1), lambda i: (i, 0)),
            pl.BlockSpec((1, F), lambda i: (0, 0)),
            pl.BlockSpec((F, F), lambda i: (0, 0)),
            pl.BlockSpec((1, F), lambda i: (0, 0)),
            pl.BlockSpec((F, 1), lambda i: (0, 0)),
            pl.BlockSpec((1, 1), lambda i: (0, 0)),
            pl.BlockSpec((F, F), lambda i: (0, 0)),
            pl.BlockSpec((1, F), lambda i: (0, 0)),
            pl.BlockSpec((F, 1), lambda i: (0, 0)),
            pl.BlockSpec((1, 1), lambda i: (0, 0)),
        ],
        out_specs=[
            pl.BlockSpec((BT, 1), lambda i: (i, 0)),
            pl.BlockSpec((1, 1), lambda i: (0, 0)),
        ],
        out_shape=[
            jax.ShapeDtypeStruct((N, 1), jnp.float32),
            jax.ShapeDtypeStruct((1, 1), jnp.float32),
        ],
        scratch_shapes=[pltpu.VMEM((1, F), jnp.float32)],
    )(s0, s1, g, dinv, b4, pw1, pb1, pw2, pb2, vw1, vb1, vw2, vb2)


def kernel(x, edge_index, pre_W1, pre_b1, pre_W2, pre_b2, conv_W, conv_b,
           pol_W1, pol_b1, pol_W2, pol_b2, val_W1, val_b1, val_W2, val_b2):
    e = edge_index.shape[1]
    src = edge_index[0].astype(jnp.int32)
    dst = edge_index[1].astype(jnp.int32)
    npad = EPAD - e
    ar = jnp.arange(npad, dtype=jnp.int32)
    srcp = jnp.concatenate([src, ar % 256]).reshape(NW, NBW, BS)
    dstp = jnp.concatenate([dst, N + ar % (NPAD - N)]).reshape(NW, NBW, BS)

    degp = _deg_kernel(dstp)
    degt = degp[:, :N].T
    g, dinv = _pre_call(x, degt, pre_W1, pre_b1.reshape(1, -1),
                        pre_W2, pre_b2.reshape(1, -1), conv_W[0])
    for i in range(3):
        spp = _scatter_kernel(srcp, dstp, g)
        g = _mid_call(spp[0, :N], spp[1, :N], g, dinv,
                      conv_b[i].reshape(1, -1), conv_W[i + 1])
    spp = _scatter_kernel(srcp, dstp, g)
    proba, value = _fin_call(
        spp[0, :N], spp[1, :N], g, dinv, conv_b[3].reshape(1, -1),
        pol_W1, pol_b1.reshape(1, -1), pol_W2, pol_b2.reshape(1, 1),
        val_W1, val_b1.reshape(1, -1), val_W2, val_b2.reshape(1, 1))
    return (proba, value)


# speedup vs baseline: 44.5212x; 1.2266x over previous
import functools

import jax
import jax.numpy as jnp
from jax import lax
from jax.experimental import pallas as pl
from jax.experimental.pallas import tpu as pltpu
from jax.experimental.pallas import tpu_sc as plsc

N = 10000
F = 16
NC, NS, LANES = 2, 16, 16
NW = NC * NS
BS = 128
NBW = 80
EPAD = NW * NBW * BS
NPAD = 10240
RPT = NPAD // NS

_HI = lax.Precision.HIGHEST

_mesh = plsc.VectorSubcoreMesh(
    core_axis_name="c", subcore_axis_name="s", num_cores=NC, num_subcores=NS
)

_sc_params = pltpu.CompilerParams(use_tc_tiling_on_sc=False)


@functools.partial(
    pl.kernel,
    out_type=jax.ShapeDtypeStruct((NC, NPAD), jnp.float32),
    mesh=_mesh,
    scratch_types=[
        pltpu.VMEM((NBW, BS), jnp.int32),
        pltpu.VMEM((BS,), jnp.float32),
        pltpu.VMEM((RPT,), jnp.float32),
        pltpu.VMEM_SHARED((NPAD,), jnp.float32),
    ],
    compiler_params=_sc_params,
)
def _deg_kernel(dst_hbm, degp_hbm, dstv, onesv, zb, acc):
    c = lax.axis_index("c")
    s = lax.axis_index("s")
    wid = c * NS + s

    def _zero(i, carry):
        zb[pl.ds(i * LANES, LANES)] = jnp.zeros((LANES,), jnp.float32)
        return carry

    lax.fori_loop(0, RPT // LANES, _zero, 0)

    def _one(i, carry):
        onesv[pl.ds(i * LANES, LANES)] = jnp.ones((LANES,), jnp.float32)
        return carry

    lax.fori_loop(0, BS // LANES, _one, 0)

    pltpu.sync_copy(zb, acc.at[pl.ds(s * RPT, RPT)])
    pltpu.sync_copy(dst_hbm.at[wid], dstv)
    plsc.subcore_barrier()

    def _scat(j, carry):
        pltpu.sync_copy(onesv, acc.at[dstv.at[j]], add=True)
        return carry

    lax.fori_loop(0, NBW, _scat, 0)
    plsc.subcore_barrier()
    pltpu.sync_copy(acc.at[pl.ds(s * RPT, RPT)],
                    degp_hbm.at[c, pl.ds(s * RPT, RPT)])


@functools.partial(
    pl.kernel,
    out_type=jax.ShapeDtypeStruct((NC, NPAD, F), jnp.float32),
    mesh=_mesh,
    scratch_types=[
        pltpu.VMEM((NBW, BS), jnp.int32),
        pltpu.VMEM((NBW, BS), jnp.int32),
        pltpu.VMEM((4, BS, F), jnp.float32),
        pltpu.VMEM((RPT, F), jnp.float32),
        pltpu.VMEM_SHARED((NPAD, F), jnp.float32),
        pltpu.VMEM_SHARED((N, F), jnp.float32),
        [pltpu.SemaphoreType.DMA] * 4,
        [pltpu.SemaphoreType.DMA] * 4,
    ],
    compiler_params=_sc_params,
)
def _scatter_kernel(src_hbm, dst_hbm, g_hbm, out_hbm,
                    srcv, dstv, rows, zb, acc, gsh, gsems, ssems):
    c = lax.axis_index("c")
    s = lax.axis_index("s")
    wid = c * NS + s

    pltpu.sync_copy(g_hbm.at[pl.ds(s * (N // NS), N // NS)],
                    gsh.at[pl.ds(s * (N // NS), N // NS)])

    def _zero(i, carry):
        zb[i, :] = jnp.zeros((LANES,), jnp.float32)
        return carry

    lax.fori_loop(0, RPT, _zero, 0)
    pltpu.sync_copy(zb, acc.at[pl.ds(s * RPT, RPT)])
    pltpu.sync_copy(src_hbm.at[wid], srcv)
    pltpu.sync_copy(dst_hbm.at[wid], dstv)
    plsc.subcore_barrier()

    def _fire_gather(j, b):
        pltpu.async_copy(gsh.at[srcv.at[j]], rows.at[b], gsems[b])

    def _fire_scatter(j, b):
        pltpu.async_copy(rows.at[b], acc.at[dstv.at[j]], ssems[b], add=True)

    def _drain_gather(j, b):
        pltpu.make_async_copy(gsh.at[srcv.at[j]], rows.at[b], gsems[b]).wait()

    def _drain_scatter(j, b):
        pltpu.make_async_copy(rows.at[b], acc.at[dstv.at[j]], ssems[b]).wait()

    _fire_gather(0, 0)
    _fire_gather(1, 1)

    def _body(jo, carry):
        j0 = 4 * jo
        for u in range(4):
            j = j0 + u
            _drain_gather(j, u)
            _fire_scatter(j, u)

            @pl.when(j >= 2)
            def _():
                _drain_scatter(j - 2, (u - 2) % 4)

            @pl.when(j + 2 < NBW)
            def _():
                _fire_gather(j + 2, (u + 2) % 4)
        return carry

    lax.fori_loop(0, NBW // 4, _body, 0)
    _drain_scatter(NBW - 2, 2)
    _drain_scatter(NBW - 1, 3)
    plsc.subcore_barrier()
    pltpu.sync_copy(acc.at[pl.ds(s * RPT, RPT)],
                    out_hbm.at[c, pl.ds(s * RPT, RPT)])


BT = 2000
GRID = N // BT


def _pre_body(x_ref, degt_ref, w1_ref, b1_ref, w2_ref, b2_ref, cw0_ref,
              g1_ref, dinv_ref):
    xb = x_ref[...]
    h = jnp.maximum(jnp.dot(xb, w1_ref[...], precision=_HI) + b1_ref[...], 0.0)
    h = jnp.dot(h, w2_ref[...], precision=_HI) + b2_ref[...]
    d = degt_ref[...]
    deg = d[:, 0:1] + d[:, 1:2] + 1.0
    dinv = lax.rsqrt(deg)
    g1_ref[...] = jnp.dot(h, cw0_ref[...], precision=_HI) * dinv
    dinv_ref[...] = dinv


def _pre_call(x, degt, w1, b1, w2, b2, cw0):
    return pl.pallas_call(
        _pre_body,
        grid=(GRID,),
        in_specs=[
            pl.BlockSpec((BT, 128), lambda i: (i, 0)),
            pl.BlockSpec((BT, 2), lambda i: (i, 0)),
            pl.BlockSpec((128, 32), lambda i: (0, 0)),
            pl.BlockSpec((1, 32), lambda i: (0, 0)),
            pl.BlockSpec((32, 64), lambda i: (0, 0)),
            pl.BlockSpec((1, 64), lambda i: (0, 0)),
            pl.BlockSpec((64, F), lambda i: (0, 0)),
        ],
        out_specs=[
            pl.BlockSpec((BT, F), lambda i: (i, 0)),
            pl.BlockSpec((BT, 1), lambda i: (i, 0)),
        ],
        out_shape=[
            jax.ShapeDtypeStruct((N, F), jnp.float32),
            jax.ShapeDtypeStruct((N, 1), jnp.float32),
        ],
    )(x, degt, w1, b1, w2, b2, cw0)


def _mid_body(s0_ref, s1_ref, g_ref, dinv_ref, b_ref, wn_ref, gout_ref):
    dinv = dinv_ref[...]
    v = (s0_ref[...] + s1_ref[...] + g_ref[...]) * dinv + b_ref[...]
    h = jnp.where(v > 0, v, jnp.exp(v) - 1.0)
    gout_ref[...] = jnp.dot(h, wn_ref[...], precision=_HI) * dinv


def _mid_call(s0, s1, g, dinv, b, wn):
    return pl.pallas_call(
        _mid_body,
        grid=(GRID,),
        in_specs=[
            pl.BlockSpec((BT, F), lambda i: (i, 0)),
            pl.BlockSpec((BT, F), lambda i: (i, 0)),
            pl.BlockSpec((BT, F), lambda i: (i, 0)),
            pl.BlockSpec((BT, 1), lambda i: (i, 0)),
            pl.BlockSpec((1, F), lambda i: (0, 0)),
            pl.BlockSpec((F, F), lambda i: (0, 0)),
        ],
        out_specs=pl.BlockSpec((BT, F), lambda i: (i, 0)),
        out_shape=jax.ShapeDtypeStruct((N, F), jnp.float32),
    )(s0, s1, g, dinv, b, wn)


def _fin_body(s0_ref, s1_ref, g_ref, dinv_ref, b4_ref,
              pw1_ref, pb1_ref, pw2_ref, pb2_ref,
              vw1_ref, vb1_ref, vw2_ref, vb2_ref,
              proba_ref, value_ref, accum):
    i = pl.program_id(0)
    dinv = dinv_ref[...]
    v = (s0_ref[...] + s1_ref[...] + g_ref[...]) * dinv + b4_ref[...]
    h = jnp.where(v > 0, v, jnp.exp(v) - 1.0)
    p = jnp.dot(h, pw1_ref[...], precision=_HI) + pb1_ref[...]
    p = jnp.where(p > 0, p, jnp.exp(p) - 1.0)
    proba_ref[...] = jnp.dot(p, pw2_ref[...], precision=_HI) + pb2_ref[...]

    bs = jnp.sum(h, axis=0, keepdims=True)

    @pl.when(i == 0)
    def _():
        accum[...] = bs

    @pl.when(i > 0)
    def _():
        accum[...] = accum[...] + bs

    @pl.when(i == GRID - 1)
    def _():
        m = accum[...] * (1.0 / N)
        vv = jnp.dot(m, vw1_ref[...], precision=_HI) + vb1_ref[...]
        vv = jnp.where(vv > 0, vv, jnp.exp(vv) - 1.0)
        value_ref[...] = jnp.dot(vv, vw2_ref[...], precision=_HI) + vb2_ref[...]


def _fin_call(s0, s1, g, dinv, b4, pw1, pb1, pw2, pb2, vw1, vb1, vw2, vb2):
    return pl.pallas_call(
        _fin_body,
        grid=(GRID,),
        in_specs=[
            pl.BlockSpec((BT, F), lambda i: (i, 0)),
            pl.BlockSpec((BT, F), lambda i: (i, 0)),
            pl.BlockSpec((BT, F), lambda i: (i, 0)),
            pl.BlockSpec((BT, 1), lambda i: (i, 0)),
            pl.BlockSpec((1, F), lambda i: (0, 0)),
            pl.BlockSpec((F, F), lambda i: (0, 0)),
            pl.BlockSpec((1, F), lambda i: (0, 0)),
            pl.BlockSpec((F, 1), lambda i: (0, 0)),
            pl.BlockSpec((1, 1), lambda i: (0, 0)),
            pl.BlockSpec((F, F), lambda i: (0, 0)),
            pl.BlockSpec((1, F), lambda i: (0, 0)),
            pl.BlockSpec((F, 1), lambda i: (0, 0)),
            pl.BlockSpec((1, 1), lambda i: (0, 0)),
        ],
        out_specs=[
            pl.BlockSpec((BT, 1), lambda i: (i, 0)),
            pl.BlockSpec((1, 1), lambda i: (0, 0)),
        ],
        out_shape=[
            jax.ShapeDtypeStruct((N, 1), jnp.float32),
            jax.ShapeDtypeStruct((1, 1), jnp.float32),
        ],
        scratch_shapes=[pltpu.VMEM((1, F), jnp.float32)],
    )(s0, s1, g, dinv, b4, pw1, pb1, pw2, pb2, vw1, vb1, vw2, vb2)


def kernel(x, edge_index, pre_W1, pre_b1, pre_W2, pre_b2, conv_W, conv_b,
           pol_W1, pol_b1, pol_W2, pol_b2, val_W1, val_b1, val_W2, val_b2):
    e = edge_index.shape[1]
    src = edge_index[0].astype(jnp.int32)
    dst = edge_index[1].astype(jnp.int32)
    npad = EPAD - e
    ar = jnp.arange(npad, dtype=jnp.int32)
    srcp = jnp.concatenate([src, ar % 256]).reshape(NW, NBW, BS)
    dstp = jnp.concatenate([dst, N + ar % (NPAD - N)]).reshape(NW, NBW, BS)

    degp = _deg_kernel(dstp)
    degt = degp[:, :N].T
    g, dinv = _pre_call(x, degt, pre_W1, pre_b1.reshape(1, -1),
                        pre_W2, pre_b2.reshape(1, -1), conv_W[0])
    for i in range(3):
        spp = _scatter_kernel(srcp, dstp, g)
        g = _mid_call(spp[0, :N], spp[1, :N], g, dinv,
                      conv_b[i].reshape(1, -1), conv_W[i + 1])
    spp = _scatter_kernel(srcp, dstp, g)
    proba, value = _fin_call(
        spp[0, :N], spp[1, :N], g, dinv, conv_b[3].reshape(1, -1),
        pol_W1, pol_b1.reshape(1, -1), pol_W2, pol_b2.reshape(1, 1),
        val_W1, val_b1.reshape(1, -1), val_W2, val_b2.reshape(1, 1))
    return (proba, value)


# speedup vs baseline: 69.2452x; 1.5553x over previous
import functools

import jax
import jax.numpy as jnp
from jax import lax
from jax.experimental import pallas as pl
from jax.experimental.pallas import tpu as pltpu
from jax.experimental.pallas import tpu_sc as plsc

N = 10000
F = 16
NC, NS, LANES = 2, 16, 16
NW = NC * NS
BS = 128
NBW = 80
EPAD = NW * NBW * BS
NPAD = 10240
RPT = NPAD // NS

_HI = lax.Precision.HIGHEST

_mesh = plsc.VectorSubcoreMesh(
    core_axis_name="c", subcore_axis_name="s", num_cores=NC, num_subcores=NS
)

_sc_params = pltpu.CompilerParams(use_tc_tiling_on_sc=False)


@functools.partial(
    pl.kernel,
    out_type=jax.ShapeDtypeStruct((NC, NPAD), jnp.float32),
    mesh=_mesh,
    scratch_types=[
        pltpu.VMEM((NBW, BS), jnp.int32),
        pltpu.VMEM((BS,), jnp.float32),
        pltpu.VMEM((RPT,), jnp.float32),
        pltpu.VMEM_SHARED((NPAD,), jnp.float32),
    ],
    compiler_params=_sc_params,
)
def _deg_kernel(dst_hbm, degp_hbm, dstv, onesv, zb, acc):
    c = lax.axis_index("c")
    s = lax.axis_index("s")
    wid = c * NS + s

    def _zero(i, carry):
        zb[pl.ds(i * LANES, LANES)] = jnp.zeros((LANES,), jnp.float32)
        return carry

    lax.fori_loop(0, RPT // LANES, _zero, 0)

    def _one(i, carry):
        onesv[pl.ds(i * LANES, LANES)] = jnp.ones((LANES,), jnp.float32)
        return carry

    lax.fori_loop(0, BS // LANES, _one, 0)

    pltpu.sync_copy(zb, acc.at[pl.ds(s * RPT, RPT)])
    pltpu.sync_copy(dst_hbm.at[wid], dstv)
    plsc.subcore_barrier()

    def _scat(j, carry):
        pltpu.sync_copy(onesv, acc.at[dstv.at[j]], add=True)
        return carry

    lax.fori_loop(0, NBW, _scat, 0)
    plsc.subcore_barrier()
    pltpu.sync_copy(acc.at[pl.ds(s * RPT, RPT)],
                    degp_hbm.at[c, pl.ds(s * RPT, RPT)])


@functools.partial(
    pl.kernel,
    out_type=jax.ShapeDtypeStruct((NC, NPAD, F), jnp.float32),
    mesh=_mesh,
    scratch_types=[
        pltpu.VMEM((NBW, BS), jnp.int32),
        pltpu.VMEM((NBW, BS), jnp.int32),
        pltpu.VMEM((4, BS, F), jnp.float32),
        pltpu.VMEM((RPT, F), jnp.float32),
        pltpu.VMEM_SHARED((NPAD, F), jnp.float32),
        pltpu.VMEM_SHARED((N, F), jnp.float32),
        [pltpu.SemaphoreType.DMA] * 4,
        [pltpu.SemaphoreType.DMA] * 4,
    ],
    compiler_params=_sc_params,
)
def _scatter_kernel(src_hbm, dst_hbm, g_hbm, out_hbm,
                    srcv, dstv, rows, zb, acc, gsh, gsems, ssems):
    c = lax.axis_index("c")
    s = lax.axis_index("s")
    wid = c * NS + s

    pltpu.sync_copy(g_hbm.at[pl.ds(s * (N // NS), N // NS)],
                    gsh.at[pl.ds(s * (N // NS), N // NS)])

    def _zero(i, carry):
        zb[i, :] = jnp.zeros((LANES,), jnp.float32)
        return carry

    lax.fori_loop(0, RPT, _zero, 0)
    pltpu.sync_copy(zb, acc.at[pl.ds(s * RPT, RPT)])
    pltpu.sync_copy(src_hbm.at[wid], srcv)
    pltpu.sync_copy(dst_hbm.at[wid], dstv)
    plsc.subcore_barrier()

    def _fire_gather(j, b):
        pltpu.async_copy(gsh.at[srcv.at[j]], rows.at[b], gsems[b])

    def _fire_scatter(j, b):
        pltpu.async_copy(rows.at[b], acc.at[dstv.at[j]], ssems[b], add=True)

    def _drain_gather(j, b):
        pltpu.make_async_copy(gsh.at[srcv.at[j]], rows.at[b], gsems[b]).wait()

    def _drain_scatter(j, b):
        pltpu.make_async_copy(rows.at[b], acc.at[dstv.at[j]], ssems[b]).wait()

    _fire_gather(0, 0)
    _fire_gather(1, 1)

    def _body(jo, carry):
        j0 = 4 * jo
        for u in range(4):
            j = j0 + u
            _drain_gather(j, u)
            _fire_scatter(j, u)

            @pl.when(j >= 2)
            def _():
                _drain_scatter(j - 2, (u - 2) % 4)

            @pl.when(j + 2 < NBW)
            def _():
                _fire_gather(j + 2, (u + 2) % 4)
        return carry

    lax.fori_loop(0, NBW // 4, _body, 0)
    _drain_scatter(NBW - 2, 2)
    _drain_scatter(NBW - 1, 3)
    plsc.subcore_barrier()
    pltpu.sync_copy(acc.at[pl.ds(s * RPT, RPT)],
                    out_hbm.at[c, pl.ds(s * RPT, RPT)])


BT = 2000
GRID = N // BT


def _pre_body(x_ref, degt_ref, w1_ref, b1_ref, w2_ref, b2_ref, cw0_ref,
              g1_ref, dinv_ref):
    xb = x_ref[...]
    h = jnp.maximum(jnp.dot(xb, w1_ref[...], precision=_HI) + b1_ref[...], 0.0)
    h = jnp.dot(h, w2_ref[...], precision=_HI) + b2_ref[...]
    d = degt_ref[...]
    deg = d[:, 0:1] + d[:, 1:2] + 1.0
    dinv = lax.rsqrt(deg)
    g1_ref[...] = jnp.dot(h, cw0_ref[...], precision=_HI) * dinv
    dinv_ref[...] = dinv


def _pre_call(x, degt, w1, b1, w2, b2, cw0):
    return pl.pallas_call(
        _pre_body,
        grid=(GRID,),
        in_specs=[
            pl.BlockSpec((BT, 128), lambda i: (i, 0)),
            pl.BlockSpec((BT, 2), lambda i: (i, 0)),
            pl.BlockSpec((128, 32), lambda i: (0, 0)),
            pl.BlockSpec((1, 32), lambda i: (0, 0)),
            pl.BlockSpec((32, 64), lambda i: (0, 0)),
            pl.BlockSpec((1, 64), lambda i: (0, 0)),
            pl.BlockSpec((64, F), lambda i: (0, 0)),
        ],
        out_specs=[
            pl.BlockSpec((BT, F), lambda i: (i, 0)),
            pl.BlockSpec((BT, 1), lambda i: (i, 0)),
        ],
        out_shape=[
            jax.ShapeDtypeStruct((N, F), jnp.float32),
            jax.ShapeDtypeStruct((N, 1), jnp.float32),
        ],
    )(x, degt, w1, b1, w2, b2, cw0)


NP = NPAD // 8
BTP = NP // GRID


def _mid_body(spp_ref, g_ref, dinv_ref, b_ref, wn_ref, gout_ref):
    dinv = dinv_ref[...]
    sp = spp_ref[...]
    v = (sp[0] + sp[1] + g_ref[...]) * dinv + b_ref[...]
    h = jnp.where(v > 0, v, jnp.exp(v) - 1.0)
    gout_ref[...] = jnp.dot(h, wn_ref[...], precision=_HI) * dinv


def _mid_call(spp, g, dinv, b, wn):
    return pl.pallas_call(
        _mid_body,
        grid=(GRID,),
        in_specs=[
            pl.BlockSpec((2, BTP, 128), lambda i: (0, i, 0)),
            pl.BlockSpec((BTP, 128), lambda i: (i, 0)),
            pl.BlockSpec((BTP, 128), lambda i: (i, 0)),
            pl.BlockSpec((1, 128), lambda i: (0, 0)),
            pl.BlockSpec((128, 128), lambda i: (0, 0)),
        ],
        out_specs=pl.BlockSpec((BTP, 128), lambda i: (i, 0)),
        out_shape=jax.ShapeDtypeStruct((NP, 128), jnp.float32),
    )(spp, g, dinv, b, wn)


def _fin_body(spp_ref, g_ref, dinv_ref, b4_ref,
              pw1_ref, pb1_ref, pw2_ref, pb2_ref,
              vw1_ref, vb1_ref, vw2_ref, vb2_ref,
              proba_ref, value_ref, accum):
    i = pl.program_id(0)
    dinv = dinv_ref[...]
    sp = spp_ref[...]
    v = (sp[0] + sp[1] + g_ref[...]) * dinv + b4_ref[...]
    h = jnp.where(v > 0, v, jnp.exp(v) - 1.0)
    p = jnp.dot(h, pw1_ref[...], precision=_HI) + pb1_ref[...]
    p = jnp.where(p > 0, p, jnp.exp(p) - 1.0)
    proba_ref[...] = jnp.dot(p, pw2_ref[...], precision=_HI) + pb2_ref[...]

    rid = lax.broadcasted_iota(jnp.int32, (BTP, 1), 0)
    hmask = jnp.where(rid < (N // 8 - i * BTP), h, 0.0)
    bs = jnp.sum(hmask, axis=0, keepdims=True)

    @pl.when(i == 0)
    def _():
        accum[...] = bs

    @pl.when(i > 0)
    def _():
        accum[...] = accum[...] + bs

    @pl.when(i == GRID - 1)
    def _():
        a = accum[...]
        m = (a[:, 0:16] + a[:, 16:32] + a[:, 32:48] + a[:, 48:64]
             + a[:, 64:80] + a[:, 80:96] + a[:, 96:112] + a[:, 112:128])
        m = m * (1.0 / N)
        vv = jnp.dot(m, vw1_ref[...], precision=_HI) + vb1_ref[...]
        vv = jnp.where(vv > 0, vv, jnp.exp(vv) - 1.0)
        value_ref[...] = jnp.dot(vv, vw2_ref[...], precision=_HI) + vb2_ref[...]


def _fin_call(spp, g, dinv, b4, pw1, pb1, pw2, pb2, vw1, vb1, vw2, vb2):
    return pl.pallas_call(
        _fin_body,
        grid=(GRID,),
        in_specs=[
            pl.BlockSpec((2, BTP, 128), lambda i: (0, i, 0)),
            pl.BlockSpec((BTP, 128), lambda i: (i, 0)),
            pl.BlockSpec((BTP, 128), lambda i: (i, 0)),
            pl.BlockSpec((1, 128), lambda i: (0, 0)),
            pl.BlockSpec((128, 128), lambda i: (0, 0)),
            pl.BlockSpec((1, 128), lambda i: (0, 0)),
            pl.BlockSpec((128, 8), lambda i: (0, 0)),
            pl.BlockSpec((1, 1), lambda i: (0, 0)),
            pl.BlockSpec((F, F), lambda i: (0, 0)),
            pl.BlockSpec((1, F), lambda i: (0, 0)),
            pl.BlockSpec((F, 1), lambda i: (0, 0)),
            pl.BlockSpec((1, 1), lambda i: (0, 0)),
        ],
        out_specs=[
            pl.BlockSpec((BTP, 8), lambda i: (i, 0)),
            pl.BlockSpec((1, 1), lambda i: (0, 0)),
        ],
        out_shape=[
            jax.ShapeDtypeStruct((NP, 8), jnp.float32),
            jax.ShapeDtypeStruct((1, 1), jnp.float32),
        ],
        scratch_shapes=[pltpu.VMEM((1, 128), jnp.float32)],
    )(spp, g, dinv, b4, pw1, pb1, pw2, pb2, vw1, vb1, vw2, vb2)


def kernel(x, edge_index, pre_W1, pre_b1, pre_W2, pre_b2, conv_W, conv_b,
           pol_W1, pol_b1, pol_W2, pol_b2, val_W1, val_b1, val_W2, val_b2):
    e = edge_index.shape[1]
    src = edge_index[0].astype(jnp.int32)
    dst = edge_index[1].astype(jnp.int32)
    npad = EPAD - e
    ar = jnp.arange(npad, dtype=jnp.int32)
    srcp = jnp.concatenate([src, ar % 256]).reshape(NW, NBW, BS)
    dstp = jnp.concatenate([dst, N + ar % (NPAD - N)]).reshape(NW, NBW, BS)

    degp = _deg_kernel(dstp)
    degt = degp[:, :N].T
    g1, dinv = _pre_call(x, degt, pre_W1, pre_b1.reshape(1, -1),
                         pre_W2, pre_b2.reshape(1, -1), conv_W[0])

    eye8 = jnp.eye(8, dtype=jnp.float32)
    pad_rows = jnp.zeros((NP - N // 8, 128), jnp.float32)
    gp = jnp.concatenate([g1.reshape(N // 8, 128), pad_rows])
    dinvp = jnp.concatenate(
        [jnp.broadcast_to(dinv, (N, F)).reshape(N // 8, 128), pad_rows])

    for i in range(3):
        spp = _scatter_kernel(srcp, dstp, gp.reshape(NPAD, F))
        gp = _mid_call(spp.reshape(2, NP, 128), gp, dinvp,
                       jnp.tile(conv_b[i], 8).reshape(1, 128),
                       jnp.kron(eye8, conv_W[i + 1]))
    spp = _scatter_kernel(srcp, dstp, gp.reshape(NPAD, F))
    probap, value = _fin_call(
        spp.reshape(2, NP, 128), gp, dinvp,
        jnp.tile(conv_b[3], 8).reshape(1, 128),
        jnp.kron(eye8, pol_W1), jnp.tile(pol_b1, 8).reshape(1, 128),
        jnp.kron(eye8, pol_W2), pol_b2.reshape(1, 1),
        val_W1, val_b1.reshape(1, -1), val_W2, val_b2.reshape(1, 1))
    proba = probap.reshape(NPAD, 1)[:N]
    return (proba, value)


# speedup vs baseline: 72.2528x; 1.0434x over previous
import functools

import jax
import jax.numpy as jnp
from jax import lax
from jax.experimental import pallas as pl
from jax.experimental.pallas import tpu as pltpu
from jax.experimental.pallas import tpu_sc as plsc

N = 10000
F = 16
NC, NS, LANES = 2, 16, 16
NW = NC * NS
BS = 128
NBW = 80
EPAD = NW * NBW * BS
NPAD = 10240
RPT = NPAD // NS

_HI = lax.Precision.HIGHEST

_mesh = plsc.VectorSubcoreMesh(
    core_axis_name="c", subcore_axis_name="s", num_cores=NC, num_subcores=NS
)

_sc_params = pltpu.CompilerParams(use_tc_tiling_on_sc=False)


@functools.partial(
    pl.kernel,
    out_type=jax.ShapeDtypeStruct((NC, NPAD), jnp.float32),
    mesh=_mesh,
    scratch_types=[
        pltpu.VMEM((NBW, BS), jnp.int32),
        pltpu.VMEM((BS,), jnp.float32),
        pltpu.VMEM((RPT,), jnp.float32),
        pltpu.VMEM_SHARED((NPAD,), jnp.float32),
    ],
    compiler_params=_sc_params,
)
def _deg_kernel(dst_hbm, degp_hbm, dstv, onesv, zb, acc):
    c = lax.axis_index("c")
    s = lax.axis_index("s")
    wid = c * NS + s

    def _zero(i, carry):
        zb[pl.ds(i * LANES, LANES)] = jnp.zeros((LANES,), jnp.float32)
        return carry

    lax.fori_loop(0, RPT // LANES, _zero, 0)

    def _one(i, carry):
        onesv[pl.ds(i * LANES, LANES)] = jnp.ones((LANES,), jnp.float32)
        return carry

    lax.fori_loop(0, BS // LANES, _one, 0)

    pltpu.sync_copy(zb, acc.at[pl.ds(s * RPT, RPT)])
    pltpu.sync_copy(dst_hbm.at[wid], dstv)
    plsc.subcore_barrier()

    def _scat(j, carry):
        pltpu.sync_copy(onesv, acc.at[dstv.at[j]], add=True)
        return carry

    lax.fori_loop(0, NBW, _scat, 0)
    plsc.subcore_barrier()
    pltpu.sync_copy(acc.at[pl.ds(s * RPT, RPT)],
                    degp_hbm.at[c, pl.ds(s * RPT, RPT)])


@functools.partial(
    pl.kernel,
    out_type=jax.ShapeDtypeStruct((NC, NPAD, F), jnp.float32),
    mesh=_mesh,
    scratch_types=[
        pltpu.VMEM((NBW, BS), jnp.int32),
        pltpu.VMEM((NBW, BS), jnp.int32),
        pltpu.VMEM((4, BS, F), jnp.float32),
        pltpu.VMEM((RPT, F), jnp.float32),
        pltpu.VMEM_SHARED((NPAD, F), jnp.float32),
        pltpu.VMEM_SHARED((N, F), jnp.float32),
        [pltpu.SemaphoreType.DMA] * 4,
        [pltpu.SemaphoreType.DMA] * 4,
    ],
    compiler_params=_sc_params,
)
def _scatter_kernel(src_hbm, dst_hbm, g_hbm, out_hbm,
                    srcv, dstv, rows, zb, acc, gsh, gsems, ssems):
    c = lax.axis_index("c")
    s = lax.axis_index("s")
    wid = c * NS + s

    pltpu.sync_copy(g_hbm.at[pl.ds(s * (N // NS), N // NS)],
                    gsh.at[pl.ds(s * (N // NS), N // NS)])

    def _zero(i, carry):
        zb[i, :] = jnp.zeros((LANES,), jnp.float32)
        return carry

    lax.fori_loop(0, RPT, _zero, 0)
    pltpu.sync_copy(zb, acc.at[pl.ds(s * RPT, RPT)])
    pltpu.sync_copy(src_hbm.at[wid], srcv)
    pltpu.sync_copy(dst_hbm.at[wid], dstv)
    plsc.subcore_barrier()

    def _fire_gather(j, b):
        pltpu.async_copy(gsh.at[srcv.at[j]], rows.at[b], gsems[b])

    def _fire_scatter(j, b):
        pltpu.async_copy(rows.at[b], acc.at[dstv.at[j]], ssems[b], add=True)

    def _drain_gather(j, b):
        pltpu.make_async_copy(gsh.at[srcv.at[j]], rows.at[b], gsems[b]).wait()

    def _drain_scatter(j, b):
        pltpu.make_async_copy(rows.at[b], acc.at[dstv.at[j]], ssems[b]).wait()

    _fire_gather(0, 0)
    _fire_gather(1, 1)

    def _body(jo, carry):
        j0 = 4 * jo
        for u in range(4):
            j = j0 + u
            _drain_gather(j, u)
            _fire_scatter(j, u)

            @pl.when(j >= 2)
            def _():
                _drain_scatter(j - 2, (u - 2) % 4)

            @pl.when(j + 2 < NBW)
            def _():
                _fire_gather(j + 2, (u + 2) % 4)
        return carry

    lax.fori_loop(0, NBW // 4, _body, 0)
    _drain_scatter(NBW - 2, 2)
    _drain_scatter(NBW - 1, 3)
    plsc.subcore_barrier()
    pltpu.sync_copy(acc.at[pl.ds(s * RPT, RPT)],
                    out_hbm.at[c, pl.ds(s * RPT, RPT)])


BT = 2000
GRID = N // BT


def _pre_body(x_ref, w1_ref, b1_ref, w2_ref, b2_ref, cw0_ref, g1_ref):
    xb = x_ref[...]
    h = jnp.maximum(jnp.dot(xb, w1_ref[...], precision=_HI) + b1_ref[...], 0.0)
    h = jnp.dot(h, w2_ref[...], precision=_HI) + b2_ref[...]
    g1_ref[...] = jnp.dot(h, cw0_ref[...], precision=_HI)


def _pre_call(x, w1, b1, w2, b2, cw0):
    return pl.pallas_call(
        _pre_body,
        grid=(GRID,),
        in_specs=[
            pl.BlockSpec((BT, 128), lambda i: (i, 0)),
            pl.BlockSpec((128, 32), lambda i: (0, 0)),
            pl.BlockSpec((1, 32), lambda i: (0, 0)),
            pl.BlockSpec((32, 64), lambda i: (0, 0)),
            pl.BlockSpec((1, 64), lambda i: (0, 0)),
            pl.BlockSpec((64, F), lambda i: (0, 0)),
        ],
        out_specs=pl.BlockSpec((BT, F), lambda i: (i, 0)),
        out_shape=jax.ShapeDtypeStruct((N, F), jnp.float32),
    )(x, w1, b1, w2, b2, cw0)


def _dinv_body(degp_ref, dinv_ref):
    d = degp_ref[...]
    deg = d[0:1, :] + d[1:2, :] + 1.0
    nid = lax.broadcasted_iota(jnp.int32, (1, NPAD), 1)
    dinv_ref[...] = jnp.where(nid < N, lax.rsqrt(deg), 0.0)


def _dinv_call(degp):
    return pl.pallas_call(
        _dinv_body,
        in_specs=[pl.BlockSpec((2, NPAD), lambda: (0, 0))],
        out_specs=pl.BlockSpec((1, NPAD), lambda: (0, 0)),
        out_shape=jax.ShapeDtypeStruct((1, NPAD), jnp.float32),
    )(degp)


NP = NPAD // 8
BTP = NP // GRID


def _mid_body(spp_ref, g_ref, dinv_ref, b_ref, wn_ref, gout_ref):
    dinv = dinv_ref[...]
    sp = spp_ref[...]
    v = (sp[0] + sp[1] + g_ref[...]) * dinv + b_ref[...]
    h = jnp.where(v > 0, v, jnp.exp(v) - 1.0)
    gout_ref[...] = jnp.dot(h, wn_ref[...], precision=_HI) * dinv


def _mid_call(spp, g, dinv, b, wn):
    return pl.pallas_call(
        _mid_body,
        grid=(GRID,),
        in_specs=[
            pl.BlockSpec((2, BTP, 128), lambda i: (0, i, 0)),
            pl.BlockSpec((BTP, 128), lambda i: (i, 0)),
            pl.BlockSpec((BTP, 128), lambda i: (i, 0)),
            pl.BlockSpec((1, 128), lambda i: (0, 0)),
            pl.BlockSpec((128, 128), lambda i: (0, 0)),
        ],
        out_specs=pl.BlockSpec((BTP, 128), lambda i: (i, 0)),
        out_shape=jax.ShapeDtypeStruct((NP, 128), jnp.float32),
    )(spp, g, dinv, b, wn)


def _fin_body(spp_ref, g_ref, dinv_ref, b4_ref,
              pw1_ref, pb1_ref, pw2_ref, pb2_ref,
              vw1_ref, vb1_ref, vw2_ref, vb2_ref,
              proba_ref, value_ref, accum):
    i = pl.program_id(0)
    dinv = dinv_ref[...]
    sp = spp_ref[...]
    v = (sp[0] + sp[1] + g_ref[...]) * dinv + b4_ref[...]
    h = jnp.where(v > 0, v, jnp.exp(v) - 1.0)
    p = jnp.dot(h, pw1_ref[...], precision=_HI) + pb1_ref[...]
    p = jnp.where(p > 0, p, jnp.exp(p) - 1.0)
    proba_ref[...] = jnp.dot(p, pw2_ref[...], precision=_HI) + pb2_ref[...]

    rid = lax.broadcasted_iota(jnp.int32, (BTP, 1), 0)
    hmask = jnp.where(rid < (N // 8 - i * BTP), h, 0.0)
    bs = jnp.sum(hmask, axis=0, keepdims=True)

    @pl.when(i == 0)
    def _():
        accum[...] = bs

    @pl.when(i > 0)
    def _():
        accum[...] = accum[...] + bs

    @pl.when(i == GRID - 1)
    def _():
        a = accum[...]
        m = (a[:, 0:16] + a[:, 16:32] + a[:, 32:48] + a[:, 48:64]
             + a[:, 64:80] + a[:, 80:96] + a[:, 96:112] + a[:, 112:128])
        m = m * (1.0 / N)
        vv = jnp.dot(m, vw1_ref[...], precision=_HI) + vb1_ref[...]
        vv = jnp.where(vv > 0, vv, jnp.exp(vv) - 1.0)
        value_ref[...] = jnp.dot(vv, vw2_ref[...], precision=_HI) + vb2_ref[...]


def _fin_call(spp, g, dinv, b4, pw1, pb1, pw2, pb2, vw1, vb1, vw2, vb2):
    return pl.pallas_call(
        _fin_body,
        grid=(GRID,),
        in_specs=[
            pl.BlockSpec((2, BTP, 128), lambda i: (0, i, 0)),
            pl.BlockSpec((BTP, 128), lambda i: (i, 0)),
            pl.BlockSpec((BTP, 128), lambda i: (i, 0)),
            pl.BlockSpec((1, 128), lambda i: (0, 0)),
            pl.BlockSpec((128, 128), lambda i: (0, 0)),
            pl.BlockSpec((1, 128), lambda i: (0, 0)),
            pl.BlockSpec((128, 8), lambda i: (0, 0)),
            pl.BlockSpec((1, 1), lambda i: (0, 0)),
            pl.BlockSpec((F, F), lambda i: (0, 0)),
            pl.BlockSpec((1, F), lambda i: (0, 0)),
            pl.BlockSpec((F, 1), lambda i: (0, 0)),
            pl.BlockSpec((1, 1), lambda i: (0, 0)),
        ],
        out_specs=[
            pl.BlockSpec((BTP, 8), lambda i: (i, 0)),
            pl.BlockSpec((1, 1), lambda i: (0, 0)),
        ],
        out_shape=[
            jax.ShapeDtypeStruct((NP, 8), jnp.float32),
            jax.ShapeDtypeStruct((1, 1), jnp.float32),
        ],
        scratch_shapes=[pltpu.VMEM((1, 128), jnp.float32)],
    )(spp, g, dinv, b4, pw1, pb1, pw2, pb2, vw1, vb1, vw2, vb2)


def kernel(x, edge_index, pre_W1, pre_b1, pre_W2, pre_b2, conv_W, conv_b,
           pol_W1, pol_b1, pol_W2, pol_b2, val_W1, val_b1, val_W2, val_b2):
    e = edge_index.shape[1]
    src = edge_index[0].astype(jnp.int32)
    dst = edge_index[1].astype(jnp.int32)
    npad = EPAD - e
    ar = jnp.arange(npad, dtype=jnp.int32)
    srcp = jnp.concatenate([src, ar % 256]).reshape(NW, NBW, BS)
    dstp = jnp.concatenate([dst, N + ar % (NPAD - N)]).reshape(NW, NBW, BS)

    degp = _deg_kernel(dstp)
    g1u = _pre_call(x, pre_W1, pre_b1.reshape(1, -1),
                    pre_W2, pre_b2.reshape(1, -1), conv_W[0])
    dinv1 = _dinv_call(degp)

    eye8 = jnp.eye(8, dtype=jnp.float32)
    dinvp = jnp.broadcast_to(dinv1.reshape(NPAD, 1), (NPAD, F)).reshape(NP, 128)
    gp = jnp.pad(g1u.reshape(N // 8, 128), ((0, NP - N // 8), (0, 0))) * dinvp

    for i in range(3):
        spp = _scatter_kernel(srcp, dstp, gp.reshape(NPAD, F))
        gp = _mid_call(spp.reshape(2, NP, 128), gp, dinvp,
                       jnp.tile(conv_b[i], 8).reshape(1, 128),
                       jnp.kron(eye8, conv_W[i + 1]))
    spp = _scatter_kernel(srcp, dstp, gp.reshape(NPAD, F))
    probap, value = _fin_call(
        spp.reshape(2, NP, 128), gp, dinvp,
        jnp.tile(conv_b[3], 8).reshape(1, 128),
        jnp.kron(eye8, pol_W1), jnp.tile(pol_b1, 8).reshape(1, 128),
        jnp.kron(eye8, pol_W2), pol_b2.reshape(1, 1),
        val_W1, val_b1.reshape(1, -1), val_W2, val_b2.reshape(1, 1))
    proba = probap.reshape(NPAD, 1)[:N]
    return (proba, value)


# speedup vs baseline: 78.1437x; 1.0815x over previous
import functools

import jax
import jax.numpy as jnp
from jax import lax
from jax.experimental import pallas as pl
from jax.experimental.pallas import tpu as pltpu
from jax.experimental.pallas import tpu_sc as plsc

N = 10000
F = 16
NC, NS, LANES = 2, 16, 16
NW = NC * NS
BS = 128
NBW = 80
EPAD = NW * NBW * BS
NPAD = 10240
RPT = NPAD // NS

_HI = lax.Precision.HIGHEST

_mesh = plsc.VectorSubcoreMesh(
    core_axis_name="c", subcore_axis_name="s", num_cores=NC, num_subcores=NS
)

_sc_params = pltpu.CompilerParams(use_tc_tiling_on_sc=False)


@functools.partial(
    pl.kernel,
    out_type=jax.ShapeDtypeStruct((NC, NPAD), jnp.float32),
    mesh=_mesh,
    scratch_types=[
        pltpu.VMEM((NBW, BS), jnp.int32),
        pltpu.VMEM((BS,), jnp.float32),
        pltpu.VMEM((RPT,), jnp.float32),
        pltpu.VMEM_SHARED((NPAD,), jnp.float32),
    ],
    compiler_params=_sc_params,
)
def _deg_kernel(dst_hbm, degp_hbm, dstv, onesv, zb, acc):
    c = lax.axis_index("c")
    s = lax.axis_index("s")
    wid = c * NS + s

    def _zero(i, carry):
        zb[pl.ds(i * LANES, LANES)] = jnp.zeros((LANES,), jnp.float32)
        return carry

    lax.fori_loop(0, RPT // LANES, _zero, 0)

    def _one(i, carry):
        onesv[pl.ds(i * LANES, LANES)] = jnp.ones((LANES,), jnp.float32)
        return carry

    lax.fori_loop(0, BS // LANES, _one, 0)

    pltpu.sync_copy(zb, acc.at[pl.ds(s * RPT, RPT)])
    pltpu.sync_copy(dst_hbm.at[wid], dstv)
    plsc.subcore_barrier()

    def _scat(j, carry):
        pltpu.sync_copy(onesv, acc.at[dstv.at[j]], add=True)
        return carry

    lax.fori_loop(0, NBW, _scat, 0)
    plsc.subcore_barrier()
    pltpu.sync_copy(acc.at[pl.ds(s * RPT, RPT)],
                    degp_hbm.at[c, pl.ds(s * RPT, RPT)])


@functools.partial(
    pl.kernel,
    out_type=jax.ShapeDtypeStruct((NC, NPAD, F), jnp.float32),
    mesh=_mesh,
    scratch_types=[
        pltpu.VMEM((NBW, BS), jnp.int32),
        pltpu.VMEM((NBW, BS), jnp.int32),
        pltpu.VMEM((4, BS, F), jnp.float32),
        pltpu.VMEM((RPT, F), jnp.float32),
        pltpu.VMEM_SHARED((NPAD, F), jnp.float32),
        pltpu.VMEM_SHARED((N, F), jnp.float32),
        [pltpu.SemaphoreType.DMA] * 4,
        [pltpu.SemaphoreType.DMA] * 4,
    ],
    compiler_params=_sc_params,
)
def _scatter_kernel(src_hbm, dst_hbm, g_hbm, out_hbm,
                    srcv, dstv, rows, zb, acc, gsh, gsems, ssems):
    c = lax.axis_index("c")
    s = lax.axis_index("s")
    wid = c * NS + s

    pltpu.sync_copy(g_hbm.at[pl.ds(s * (N // NS), N // NS)],
                    gsh.at[pl.ds(s * (N // NS), N // NS)])

    def _zero(i, carry):
        zb[i, :] = jnp.zeros((LANES,), jnp.float32)
        return carry

    lax.fori_loop(0, RPT, _zero, 0)
    pltpu.sync_copy(zb, acc.at[pl.ds(s * RPT, RPT)])
    pltpu.sync_copy(src_hbm.at[wid], srcv)
    pltpu.sync_copy(dst_hbm.at[wid], dstv)
    plsc.subcore_barrier()

    def _fire_gather(j, b):
        pltpu.async_copy(gsh.at[srcv.at[j]], rows.at[b], gsems[b])

    def _fire_scatter(j, b):
        pltpu.async_copy(rows.at[b], acc.at[dstv.at[j]], ssems[b], add=True)

    def _drain_gather(j, b):
        pltpu.make_async_copy(gsh.at[srcv.at[j]], rows.at[b], gsems[b]).wait()

    def _drain_scatter(j, b):
        pltpu.make_async_copy(rows.at[b], acc.at[dstv.at[j]], ssems[b]).wait()

    _fire_gather(0, 0)
    _fire_gather(1, 1)

    def _body(jo, carry):
        j0 = 4 * jo
        for u in range(4):
            j = j0 + u
            _drain_gather(j, u)
            _fire_scatter(j, u)

            @pl.when(j >= 2)
            def _():
                _drain_scatter(j - 2, (u - 2) % 4)

            @pl.when(j + 2 < NBW)
            def _():
                _fire_gather(j + 2, (u + 2) % 4)
        return carry

    lax.fori_loop(0, NBW // 4, _body, 0)
    _drain_scatter(NBW - 2, 2)
    _drain_scatter(NBW - 1, 3)
    plsc.subcore_barrier()
    pltpu.sync_copy(acc.at[pl.ds(s * RPT, RPT)],
                    out_hbm.at[c, pl.ds(s * RPT, RPT)])


BT = 2000
GRID = N // BT


def _pre_body(x_ref, w1_ref, b1_ref, w2_ref, b2_ref, cw0_ref, g1_ref):
    xb = x_ref[...]
    h = jnp.maximum(jnp.dot(xb, w1_ref[...]) + b1_ref[...], 0.0)
    h = jnp.dot(h, w2_ref[...]) + b2_ref[...]
    g1_ref[...] = jnp.dot(h, cw0_ref[...])


def _pre_call(x, w1, b1, w2, b2, cw0):
    return pl.pallas_call(
        _pre_body,
        grid=(GRID,),
        in_specs=[
            pl.BlockSpec((BT, 128), lambda i: (i, 0)),
            pl.BlockSpec((128, 32), lambda i: (0, 0)),
            pl.BlockSpec((1, 32), lambda i: (0, 0)),
            pl.BlockSpec((32, 64), lambda i: (0, 0)),
            pl.BlockSpec((1, 64), lambda i: (0, 0)),
            pl.BlockSpec((64, F), lambda i: (0, 0)),
        ],
        out_specs=pl.BlockSpec((BT, F), lambda i: (i, 0)),
        out_shape=jax.ShapeDtypeStruct((N, F), jnp.float32),
    )(x, w1, b1, w2, b2, cw0)


def _dinv_body(degp_ref, dinv_ref):
    nr = NPAD // 128
    d = degp_ref[...]
    deg = d[0:nr, :] + d[nr:2 * nr, :] + 1.0
    nid = (lax.broadcasted_iota(jnp.int32, (nr, 128), 0) * 128
           + lax.broadcasted_iota(jnp.int32, (nr, 128), 1))
    dinv_ref[...] = jnp.where(nid < N, lax.rsqrt(deg), 0.0)


def _dinv_call(degp160):
    nr = NPAD // 128
    return pl.pallas_call(
        _dinv_body,
        in_specs=[pl.BlockSpec((2 * nr, 128), lambda: (0, 0))],
        out_specs=pl.BlockSpec((nr, 128), lambda: (0, 0)),
        out_shape=jax.ShapeDtypeStruct((nr, 128), jnp.float32),
    )(degp160)


NP = NPAD // 8
BTP = NP // GRID


def _mid_body(spp_ref, g_ref, dinv_ref, b_ref, wn_ref, gout_ref):
    dinv = dinv_ref[...]
    sp = spp_ref[...]
    v = (sp[0] + sp[1] + g_ref[...]) * dinv + b_ref[...]
    h = jnp.where(v > 0, v, jnp.exp(v) - 1.0)
    gout_ref[...] = jnp.dot(h, wn_ref[...], precision=_HI) * dinv


def _mid_call(spp, g, dinv, b, wn):
    return pl.pallas_call(
        _mid_body,
        grid=(GRID,),
        in_specs=[
            pl.BlockSpec((2, BTP, 128), lambda i: (0, i, 0)),
            pl.BlockSpec((BTP, 128), lambda i: (i, 0)),
            pl.BlockSpec((BTP, 128), lambda i: (i, 0)),
            pl.BlockSpec((1, 128), lambda i: (0, 0)),
            pl.BlockSpec((128, 128), lambda i: (0, 0)),
        ],
        out_specs=pl.BlockSpec((BTP, 128), lambda i: (i, 0)),
        out_shape=jax.ShapeDtypeStruct((NP, 128), jnp.float32),
    )(spp, g, dinv, b, wn)


def _fin_body(spp_ref, g_ref, dinv_ref, b4_ref,
              pw1_ref, pb1_ref, pw2_ref, pb2_ref,
              vw1_ref, vb1_ref, vw2_ref, vb2_ref,
              proba_ref, value_ref, accum):
    i = pl.program_id(0)
    dinv = dinv_ref[...]
    sp = spp_ref[...]
    v = (sp[0] + sp[1] + g_ref[...]) * dinv + b4_ref[...]
    h = jnp.where(v > 0, v, jnp.exp(v) - 1.0)
    p = jnp.dot(h, pw1_ref[...], precision=_HI) + pb1_ref[...]
    p = jnp.where(p > 0, p, jnp.exp(p) - 1.0)
    proba_ref[...] = jnp.dot(p, pw2_ref[...], precision=_HI) + pb2_ref[...]

    rid = lax.broadcasted_iota(jnp.int32, (BTP, 1), 0)
    hmask = jnp.where(rid < (N // 8 - i * BTP), h, 0.0)
    bs = jnp.sum(hmask, axis=0, keepdims=True)

    @pl.when(i == 0)
    def _():
        accum[...] = bs

    @pl.when(i > 0)
    def _():
        accum[...] = accum[...] + bs

    @pl.when(i == GRID - 1)
    def _():
        a = accum[...]
        m = (a[:, 0:16] + a[:, 16:32] + a[:, 32:48] + a[:, 48:64]
             + a[:, 64:80] + a[:, 80:96] + a[:, 96:112] + a[:, 112:128])
        m = m * (1.0 / N)
        vv = jnp.dot(m, vw1_ref[...], precision=_HI) + vb1_ref[...]
        vv = jnp.where(vv > 0, vv, jnp.exp(vv) - 1.0)
        value_ref[...] = jnp.dot(vv, vw2_ref[...], precision=_HI) + vb2_ref[...]


def _fin_call(spp, g, dinv, b4, pw1, pb1, pw2, pb2, vw1, vb1, vw2, vb2):
    return pl.pallas_call(
        _fin_body,
        grid=(GRID,),
        in_specs=[
            pl.BlockSpec((2, BTP, 128), lambda i: (0, i, 0)),
            pl.BlockSpec((BTP, 128), lambda i: (i, 0)),
            pl.BlockSpec((BTP, 128), lambda i: (i, 0)),
            pl.BlockSpec((1, 128), lambda i: (0, 0)),
            pl.BlockSpec((128, 128), lambda i: (0, 0)),
            pl.BlockSpec((1, 128), lambda i: (0, 0)),
            pl.BlockSpec((128, 8), lambda i: (0, 0)),
            pl.BlockSpec((1, 1), lambda i: (0, 0)),
            pl.BlockSpec((F, F), lambda i: (0, 0)),
            pl.BlockSpec((1, F), lambda i: (0, 0)),
            pl.BlockSpec((F, 1), lambda i: (0, 0)),
            pl.BlockSpec((1, 1), lambda i: (0, 0)),
        ],
        out_specs=[
            pl.BlockSpec((BTP, 8), lambda i: (i, 0)),
            pl.BlockSpec((1, 1), lambda i: (0, 0)),
        ],
        out_shape=[
            jax.ShapeDtypeStruct((NP, 8), jnp.float32),
            jax.ShapeDtypeStruct((1, 1), jnp.float32),
        ],
        scratch_shapes=[pltpu.VMEM((1, 128), jnp.float32)],
    )(spp, g, dinv, b4, pw1, pb1, pw2, pb2, vw1, vb1, vw2, vb2)


def kernel(x, edge_index, pre_W1, pre_b1, pre_W2, pre_b2, conv_W, conv_b,
           pol_W1, pol_b1, pol_W2, pol_b2, val_W1, val_b1, val_W2, val_b2):
    e = edge_index.shape[1]
    eflat = edge_index.astype(jnp.int32).reshape(-1)
    src = eflat[:e]
    dst = eflat[e:]
    npad = EPAD - e
    ar = jnp.arange(npad, dtype=jnp.int32)
    srcp = jnp.concatenate([src, ar % 256]).reshape(NW, NBW, BS)
    dstp = jnp.concatenate([dst, N + ar % (NPAD - N)]).reshape(NW, NBW, BS)

    degp = _deg_kernel(dstp)
    g1u = _pre_call(x, pre_W1, pre_b1.reshape(1, -1),
                    pre_W2, pre_b2.reshape(1, -1), conv_W[0])
    dinv1 = _dinv_call(degp.reshape(-1, 128))

    eye8 = jnp.eye(8, dtype=jnp.float32)
    dinvp = jnp.broadcast_to(dinv1.reshape(NPAD, 1), (NPAD, F)).reshape(NP, 128)
    gp = jnp.pad(g1u.reshape(N // 8, 128), ((0, NP - N // 8), (0, 0))) * dinvp

    for i in range(3):
        spp = _scatter_kernel(srcp, dstp, gp.reshape(NPAD, F))
        gp = _mid_call(spp.reshape(2, NP, 128), gp, dinvp,
                       jnp.tile(conv_b[i], 8).reshape(1, 128),
                       jnp.kron(eye8, conv_W[i + 1]))
    spp = _scatter_kernel(srcp, dstp, gp.reshape(NPAD, F))
    probap, value = _fin_call(
        spp.reshape(2, NP, 128), gp, dinvp,
        jnp.tile(conv_b[3], 8).reshape(1, 128),
        jnp.kron(eye8, pol_W1), jnp.tile(pol_b1, 8).reshape(1, 128),
        jnp.kron(eye8, pol_W2), pol_b2.reshape(1, 1),
        val_W1, val_b1.reshape(1, -1), val_W2, val_b2.reshape(1, 1))
    proba = probap.reshape(NPAD, 1)[:N]
    return (proba, value)


# speedup vs baseline: 81.0090x; 1.0367x over previous
import functools

import jax
import jax.numpy as jnp
from jax import lax
from jax.experimental import pallas as pl
from jax.experimental.pallas import tpu as pltpu
from jax.experimental.pallas import tpu_sc as plsc

N = 10000
F = 16
NC, NS, LANES = 2, 16, 16
NW = NC * NS
BS = 256
NBW = 40
EPAD = NW * NBW * BS
NPAD = 10240
RPT = NPAD // NS

_HI = lax.Precision.HIGHEST

_mesh = plsc.VectorSubcoreMesh(
    core_axis_name="c", subcore_axis_name="s", num_cores=NC, num_subcores=NS
)

_sc_params = pltpu.CompilerParams(use_tc_tiling_on_sc=False)


@functools.partial(
    pl.kernel,
    out_type=jax.ShapeDtypeStruct((NC, NPAD), jnp.float32),
    mesh=_mesh,
    scratch_types=[
        pltpu.VMEM((NBW, BS), jnp.int32),
        pltpu.VMEM((BS,), jnp.float32),
        pltpu.VMEM((RPT,), jnp.float32),
        pltpu.VMEM_SHARED((NPAD,), jnp.float32),
    ],
    compiler_params=_sc_params,
)
def _deg_kernel(dst_hbm, degp_hbm, dstv, onesv, zb, acc):
    c = lax.axis_index("c")
    s = lax.axis_index("s")
    wid = c * NS + s

    def _zero(i, carry):
        zb[pl.ds(i * LANES, LANES)] = jnp.zeros((LANES,), jnp.float32)
        return carry

    lax.fori_loop(0, RPT // LANES, _zero, 0)

    def _one(i, carry):
        onesv[pl.ds(i * LANES, LANES)] = jnp.ones((LANES,), jnp.float32)
        return carry

    lax.fori_loop(0, BS // LANES, _one, 0)

    pltpu.sync_copy(zb, acc.at[pl.ds(s * RPT, RPT)])
    pltpu.sync_copy(dst_hbm.at[wid], dstv)
    plsc.subcore_barrier()

    def _scat(j, carry):
        pltpu.sync_copy(onesv, acc.at[dstv.at[j]], add=True)
        return carry

    lax.fori_loop(0, NBW, _scat, 0)
    plsc.subcore_barrier()
    pltpu.sync_copy(acc.at[pl.ds(s * RPT, RPT)],
                    degp_hbm.at[c, pl.ds(s * RPT, RPT)])


@functools.partial(
    pl.kernel,
    out_type=jax.ShapeDtypeStruct((NC, NPAD, F), jnp.float32),
    mesh=_mesh,
    scratch_types=[
        pltpu.VMEM((NBW, BS), jnp.int32),
        pltpu.VMEM((NBW, BS), jnp.int32),
        pltpu.VMEM((4, BS, F), jnp.float32),
        pltpu.VMEM((RPT, F), jnp.float32),
        pltpu.VMEM_SHARED((NPAD, F), jnp.float32),
        pltpu.VMEM_SHARED((N, F), jnp.float32),
        [pltpu.SemaphoreType.DMA] * 4,
        [pltpu.SemaphoreType.DMA] * 4,
    ],
    compiler_params=_sc_params,
)
def _scatter_kernel(src_hbm, dst_hbm, g_hbm, out_hbm,
                    srcv, dstv, rows, zb, acc, gsh, gsems, ssems):
    c = lax.axis_index("c")
    s = lax.axis_index("s")
    wid = c * NS + s

    pltpu.sync_copy(g_hbm.at[pl.ds(s * (N // NS), N // NS)],
                    gsh.at[pl.ds(s * (N // NS), N // NS)])

    def _zero(i, carry):
        zb[i, :] = jnp.zeros((LANES,), jnp.float32)
        return carry

    lax.fori_loop(0, RPT, _zero, 0)
    pltpu.sync_copy(zb, acc.at[pl.ds(s * RPT, RPT)])
    pltpu.sync_copy(src_hbm.at[wid], srcv)
    pltpu.sync_copy(dst_hbm.at[wid], dstv)
    plsc.subcore_barrier()

    def _fire_gather(j, b):
        pltpu.async_copy(gsh.at[srcv.at[j]], rows.at[b], gsems[b])

    def _fire_scatter(j, b):
        pltpu.async_copy(rows.at[b], acc.at[dstv.at[j]], ssems[b], add=True)

    def _drain_gather(j, b):
        pltpu.make_async_copy(gsh.at[srcv.at[j]], rows.at[b], gsems[b]).wait()

    def _drain_scatter(j, b):
        pltpu.make_async_copy(rows.at[b], acc.at[dstv.at[j]], ssems[b]).wait()

    _fire_gather(0, 0)
    _fire_gather(1, 1)

    def _body(jo, carry):
        j0 = 4 * jo
        for u in range(4):
            j = j0 + u
            _drain_gather(j, u)
            _fire_scatter(j, u)

            @pl.when(j >= 2)
            def _():
                _drain_scatter(j - 2, (u - 2) % 4)

            @pl.when(j + 2 < NBW)
            def _():
                _fire_gather(j + 2, (u + 2) % 4)
        return carry

    lax.fori_loop(0, NBW // 4, _body, 0)
    _drain_scatter(NBW - 2, 2)
    _drain_scatter(NBW - 1, 3)
    plsc.subcore_barrier()
    pltpu.sync_copy(acc.at[pl.ds(s * RPT, RPT)],
                    out_hbm.at[c, pl.ds(s * RPT, RPT)])


BT = 2000
GRID = N // BT


def _pre_body(x_ref, w1_ref, b1_ref, w2_ref, b2_ref, cw0_ref, g1_ref):
    xb = x_ref[...]
    h = jnp.maximum(jnp.dot(xb, w1_ref[...]) + b1_ref[...], 0.0)
    h = jnp.dot(h, w2_ref[...]) + b2_ref[...]
    g1_ref[...] = jnp.dot(h, cw0_ref[...])


def _pre_call(x, w1, b1, w2, b2, cw0):
    return pl.pallas_call(
        _pre_body,
        grid=(GRID,),
        in_specs=[
            pl.BlockSpec((BT, 128), lambda i: (i, 0)),
            pl.BlockSpec((128, 32), lambda i: (0, 0)),
            pl.BlockSpec((1, 32), lambda i: (0, 0)),
            pl.BlockSpec((32, 64), lambda i: (0, 0)),
            pl.BlockSpec((1, 64), lambda i: (0, 0)),
            pl.BlockSpec((64, F), lambda i: (0, 0)),
        ],
        out_specs=pl.BlockSpec((BT, F), lambda i: (i, 0)),
        out_shape=jax.ShapeDtypeStruct((N, F), jnp.float32),
    )(x, w1, b1, w2, b2, cw0)


def _dinv_body(degp_ref, dinv_ref):
    nr = NPAD // 128
    d = degp_ref[...]
    deg = d[0:nr, :] + d[nr:2 * nr, :] + 1.0
    nid = (lax.broadcasted_iota(jnp.int32, (nr, 128), 0) * 128
           + lax.broadcasted_iota(jnp.int32, (nr, 128), 1))
    dinv_ref[...] = jnp.where(nid < N, lax.rsqrt(deg), 0.0)


def _dinv_call(degp160):
    nr = NPAD // 128
    return pl.pallas_call(
        _dinv_body,
        in_specs=[pl.BlockSpec((2 * nr, 128), lambda: (0, 0))],
        out_specs=pl.BlockSpec((nr, 128), lambda: (0, 0)),
        out_shape=jax.ShapeDtypeStruct((nr, 128), jnp.float32),
    )(degp160)


NP = NPAD // 8
BTP = NP // GRID


def _mid_body(spp_ref, g_ref, dinv_ref, b_ref, wn_ref, gout_ref):
    dinv = dinv_ref[...]
    sp = spp_ref[...]
    v = (sp[0] + sp[1] + g_ref[...]) * dinv + b_ref[...]
    h = jnp.where(v > 0, v, jnp.exp(v) - 1.0)
    gout_ref[...] = jnp.dot(h, wn_ref[...], precision=_HI) * dinv


def _mid_call(spp, g, dinv, b, wn):
    return pl.pallas_call(
        _mid_body,
        grid=(GRID,),
        in_specs=[
            pl.BlockSpec((2, BTP, 128), lambda i: (0, i, 0)),
            pl.BlockSpec((BTP, 128), lambda i: (i, 0)),
            pl.BlockSpec((BTP, 128), lambda i: (i, 0)),
            pl.BlockSpec((1, 128), lambda i: (0, 0)),
            pl.BlockSpec((128, 128), lambda i: (0, 0)),
        ],
        out_specs=pl.BlockSpec((BTP, 128), lambda i: (i, 0)),
        out_shape=jax.ShapeDtypeStruct((NP, 128), jnp.float32),
    )(spp, g, dinv, b, wn)


def _fin_body(spp_ref, g_ref, dinv_ref, b4_ref,
              pw1_ref, pb1_ref, pw2_ref, pb2_ref,
              vw1_ref, vb1_ref, vw2_ref, vb2_ref,
              proba_ref, value_ref, accum):
    i = pl.program_id(0)
    dinv = dinv_ref[...]
    sp = spp_ref[...]
    v = (sp[0] + sp[1] + g_ref[...]) * dinv + b4_ref[...]
    h = jnp.where(v > 0, v, jnp.exp(v) - 1.0)
    p = jnp.dot(h, pw1_ref[...], precision=_HI) + pb1_ref[...]
    p = jnp.where(p > 0, p, jnp.exp(p) - 1.0)
    proba_ref[...] = jnp.dot(p, pw2_ref[...], precision=_HI) + pb2_ref[...]

    rid = lax.broadcasted_iota(jnp.int32, (BTP, 1), 0)
    hmask = jnp.where(rid < (N // 8 - i * BTP), h, 0.0)
    bs = jnp.sum(hmask, axis=0, keepdims=True)

    @pl.when(i == 0)
    def _():
        accum[...] = bs

    @pl.when(i > 0)
    def _():
        accum[...] = accum[...] + bs

    @pl.when(i == GRID - 1)
    def _():
        a = accum[...]
        m = (a[:, 0:16] + a[:, 16:32] + a[:, 32:48] + a[:, 48:64]
             + a[:, 64:80] + a[:, 80:96] + a[:, 96:112] + a[:, 112:128])
        m = m * (1.0 / N)
        vv = jnp.dot(m, vw1_ref[...], precision=_HI) + vb1_ref[...]
        vv = jnp.where(vv > 0, vv, jnp.exp(vv) - 1.0)
        value_ref[...] = jnp.dot(vv, vw2_ref[...], precision=_HI) + vb2_ref[...]


def _fin_call(spp, g, dinv, b4, pw1, pb1, pw2, pb2, vw1, vb1, vw2, vb2):
    return pl.pallas_call(
        _fin_body,
        grid=(GRID,),
        in_specs=[
            pl.BlockSpec((2, BTP, 128), lambda i: (0, i, 0)),
            pl.BlockSpec((BTP, 128), lambda i: (i, 0)),
            pl.BlockSpec((BTP, 128), lambda i: (i, 0)),
            pl.BlockSpec((1, 128), lambda i: (0, 0)),
            pl.BlockSpec((128, 128), lambda i: (0, 0)),
            pl.BlockSpec((1, 128), lambda i: (0, 0)),
            pl.BlockSpec((128, 8), lambda i: (0, 0)),
            pl.BlockSpec((1, 1), lambda i: (0, 0)),
            pl.BlockSpec((F, F), lambda i: (0, 0)),
            pl.BlockSpec((1, F), lambda i: (0, 0)),
            pl.BlockSpec((F, 1), lambda i: (0, 0)),
            pl.BlockSpec((1, 1), lambda i: (0, 0)),
        ],
        out_specs=[
            pl.BlockSpec((BTP, 8), lambda i: (i, 0)),
            pl.BlockSpec((1, 1), lambda i: (0, 0)),
        ],
        out_shape=[
            jax.ShapeDtypeStruct((NP, 8), jnp.float32),
            jax.ShapeDtypeStruct((1, 1), jnp.float32),
        ],
        scratch_shapes=[pltpu.VMEM((1, 128), jnp.float32)],
    )(spp, g, dinv, b4, pw1, pb1, pw2, pb2, vw1, vb1, vw2, vb2)


def kernel(x, edge_index, pre_W1, pre_b1, pre_W2, pre_b2, conv_W, conv_b,
           pol_W1, pol_b1, pol_W2, pol_b2, val_W1, val_b1, val_W2, val_b2):
    e = edge_index.shape[1]
    eflat = edge_index.astype(jnp.int32).reshape(-1)
    src = eflat[:e]
    dst = eflat[e:]
    npad = EPAD - e
    ar = jnp.arange(npad, dtype=jnp.int32)
    srcp = jnp.concatenate([src, ar % 256]).reshape(NW, NBW, BS)
    dstp = jnp.concatenate([dst, N + ar % (NPAD - N)]).reshape(NW, NBW, BS)

    degp = _deg_kernel(dstp)
    g1u = _pre_call(x, pre_W1, pre_b1.reshape(1, -1),
                    pre_W2, pre_b2.reshape(1, -1), conv_W[0])
    dinv1 = _dinv_call(degp.reshape(-1, 128))

    eye8 = jnp.eye(8, dtype=jnp.float32)
    dinvp = jnp.broadcast_to(dinv1.reshape(NPAD, 1), (NPAD, F)).reshape(NP, 128)
    gp = jnp.pad(g1u.reshape(N // 8, 128), ((0, NP - N // 8), (0, 0))) * dinvp

    for i in range(3):
        spp = _scatter_kernel(srcp, dstp, gp.reshape(NPAD, F))
        gp = _mid_call(spp.reshape(2, NP, 128), gp, dinvp,
                       jnp.tile(conv_b[i], 8).reshape(1, 128),
                       jnp.kron(eye8, conv_W[i + 1]))
    spp = _scatter_kernel(srcp, dstp, gp.reshape(NPAD, F))
    probap, value = _fin_call(
        spp.reshape(2, NP, 128), gp, dinvp,
        jnp.tile(conv_b[3], 8).reshape(1, 128),
        jnp.kron(eye8, pol_W1), jnp.tile(pol_b1, 8).reshape(1, 128),
        jnp.kron(eye8, pol_W2), pol_b2.reshape(1, 1),
        val_W1, val_b1.reshape(1, -1), val_W2, val_b2.reshape(1, 1))
    proba = probap.reshape(NPAD, 1)[:N]
    return (proba, value)


# speedup vs baseline: 84.0980x; 1.0381x over previous
import functools

import jax
import jax.numpy as jnp
from jax import lax
from jax.experimental import pallas as pl
from jax.experimental.pallas import tpu as pltpu
from jax.experimental.pallas import tpu_sc as plsc

N = 10000
F = 16
NC, NS, LANES = 2, 16, 16
NW = NC * NS
BS = 512
NBW = 20
EPAD = NW * NBW * BS
NPAD = 10240
RPT = NPAD // NS

_HI = lax.Precision.HIGHEST

_mesh = plsc.VectorSubcoreMesh(
    core_axis_name="c", subcore_axis_name="s", num_cores=NC, num_subcores=NS
)

_sc_params = pltpu.CompilerParams(use_tc_tiling_on_sc=False)


@functools.partial(
    pl.kernel,
    out_type=jax.ShapeDtypeStruct((NC, NPAD), jnp.float32),
    mesh=_mesh,
    scratch_types=[
        pltpu.VMEM((NBW, BS), jnp.int32),
        pltpu.VMEM((BS,), jnp.float32),
        pltpu.VMEM((RPT,), jnp.float32),
        pltpu.VMEM_SHARED((NPAD,), jnp.float32),
    ],
    compiler_params=_sc_params,
)
def _deg_kernel(dst_hbm, degp_hbm, dstv, onesv, zb, acc):
    c = lax.axis_index("c")
    s = lax.axis_index("s")
    wid = c * NS + s

    def _zero(i, carry):
        zb[pl.ds(i * LANES, LANES)] = jnp.zeros((LANES,), jnp.float32)
        return carry

    lax.fori_loop(0, RPT // LANES, _zero, 0)

    def _one(i, carry):
        onesv[pl.ds(i * LANES, LANES)] = jnp.ones((LANES,), jnp.float32)
        return carry

    lax.fori_loop(0, BS // LANES, _one, 0)

    pltpu.sync_copy(zb, acc.at[pl.ds(s * RPT, RPT)])
    pltpu.sync_copy(dst_hbm.at[wid], dstv)
    plsc.subcore_barrier()

    def _scat(j, carry):
        pltpu.sync_copy(onesv, acc.at[dstv.at[j]], add=True)
        return carry

    lax.fori_loop(0, NBW, _scat, 0)
    plsc.subcore_barrier()
    pltpu.sync_copy(acc.at[pl.ds(s * RPT, RPT)],
                    degp_hbm.at[c, pl.ds(s * RPT, RPT)])


@functools.partial(
    pl.kernel,
    out_type=jax.ShapeDtypeStruct((NC, NPAD, F), jnp.float32),
    mesh=_mesh,
    scratch_types=[
        pltpu.VMEM((NBW, BS), jnp.int32),
        pltpu.VMEM((NBW, BS), jnp.int32),
        pltpu.VMEM((4, BS, F), jnp.float32),
        pltpu.VMEM_SHARED((NPAD, F), jnp.float32),
        pltpu.VMEM_SHARED((N, F), jnp.float32),
        [pltpu.SemaphoreType.DMA] * 4,
        [pltpu.SemaphoreType.DMA] * 4,
    ],
    compiler_params=_sc_params,
)
def _scatter_kernel(src_hbm, dst_hbm, g_hbm, zeros_hbm, out_hbm,
                    srcv, dstv, rows, acc, gsh, gsems, ssems):
    c = lax.axis_index("c")
    s = lax.axis_index("s")
    wid = c * NS + s

    pltpu.sync_copy(g_hbm.at[pl.ds(s * (N // NS), N // NS)],
                    gsh.at[pl.ds(s * (N // NS), N // NS)])
    pltpu.sync_copy(zeros_hbm.at[pl.ds(s * RPT, RPT)],
                    acc.at[pl.ds(s * RPT, RPT)])
    pltpu.sync_copy(src_hbm.at[wid], srcv)
    pltpu.sync_copy(dst_hbm.at[wid], dstv)
    plsc.subcore_barrier()

    def _fire_gather(j, b):
        pltpu.async_copy(gsh.at[srcv.at[j]], rows.at[b], gsems[b])

    def _fire_scatter(j, b):
        pltpu.async_copy(rows.at[b], acc.at[dstv.at[j]], ssems[b], add=True)

    def _drain_gather(j, b):
        pltpu.make_async_copy(gsh.at[srcv.at[j]], rows.at[b], gsems[b]).wait()

    def _drain_scatter(j, b):
        pltpu.make_async_copy(rows.at[b], acc.at[dstv.at[j]], ssems[b]).wait()

    _fire_gather(0, 0)
    _fire_gather(1, 1)

    def _body(jo, carry):
        j0 = 4 * jo
        for u in range(4):
            j = j0 + u
            _drain_gather(j, u)
            _fire_scatter(j, u)

            @pl.when(j >= 2)
            def _():
                _drain_scatter(j - 2, (u - 2) % 4)

            @pl.when(j + 2 < NBW)
            def _():
                _fire_gather(j + 2, (u + 2) % 4)
        return carry

    lax.fori_loop(0, NBW // 4, _body, 0)
    _drain_scatter(NBW - 2, 2)
    _drain_scatter(NBW - 1, 3)
    plsc.subcore_barrier()
    pltpu.sync_copy(acc.at[pl.ds(s * RPT, RPT)],
                    out_hbm.at[c, pl.ds(s * RPT, RPT)])


BT = 2000
GRID = N // BT


def _pre_body(x_ref, w1_ref, b1_ref, w2_ref, b2_ref, cw0_ref, g1_ref):
    xb = x_ref[...]
    h = jnp.maximum(jnp.dot(xb, w1_ref[...]) + b1_ref[...], 0.0)
    h = jnp.dot(h, w2_ref[...]) + b2_ref[...]
    g1_ref[...] = jnp.dot(h, cw0_ref[...])


def _pre_call(x, w1, b1, w2, b2, cw0):
    return pl.pallas_call(
        _pre_body,
        grid=(GRID,),
        in_specs=[
            pl.BlockSpec((BT, 128), lambda i: (i, 0)),
            pl.BlockSpec((128, 32), lambda i: (0, 0)),
            pl.BlockSpec((1, 32), lambda i: (0, 0)),
            pl.BlockSpec((32, 64), lambda i: (0, 0)),
            pl.BlockSpec((1, 64), lambda i: (0, 0)),
            pl.BlockSpec((64, F), lambda i: (0, 0)),
        ],
        out_specs=pl.BlockSpec((BT, F), lambda i: (i, 0)),
        out_shape=jax.ShapeDtypeStruct((N, F), jnp.float32),
    )(x, w1, b1, w2, b2, cw0)


def _dinv_body(degp_ref, dinv_ref):
    nr = NPAD // 128
    d = degp_ref[...]
    deg = d[0:nr, :] + d[nr:2 * nr, :] + 1.0
    nid = (lax.broadcasted_iota(jnp.int32, (nr, 128), 0) * 128
           + lax.broadcasted_iota(jnp.int32, (nr, 128), 1))
    dinv_ref[...] = jnp.where(nid < N, lax.rsqrt(deg), 0.0)


def _dinv_call(degp160):
    nr = NPAD // 128
    return pl.pallas_call(
        _dinv_body,
        in_specs=[pl.BlockSpec((2 * nr, 128), lambda: (0, 0))],
        out_specs=pl.BlockSpec((nr, 128), lambda: (0, 0)),
        out_shape=jax.ShapeDtypeStruct((nr, 128), jnp.float32),
    )(degp160)


NP = NPAD // 8
BTP = NP // GRID


def _mid_body(spp_ref, g_ref, dinv_ref, b_ref, wn_ref, gout_ref):
    dinv = dinv_ref[...]
    sp = spp_ref[...]
    v = (sp[0] + sp[1] + g_ref[...]) * dinv + b_ref[...]
    h = jnp.where(v > 0, v, jnp.exp(v) - 1.0)
    gout_ref[...] = jnp.dot(h, wn_ref[...], precision=_HI) * dinv


def _mid_call(spp, g, dinv, b, wn):
    return pl.pallas_call(
        _mid_body,
        grid=(GRID,),
        in_specs=[
            pl.BlockSpec((2, BTP, 128), lambda i: (0, i, 0)),
            pl.BlockSpec((BTP, 128), lambda i: (i, 0)),
            pl.BlockSpec((BTP, 128), lambda i: (i, 0)),
            pl.BlockSpec((1, 128), lambda i: (0, 0)),
            pl.BlockSpec((128, 128), lambda i: (0, 0)),
        ],
        out_specs=pl.BlockSpec((BTP, 128), lambda i: (i, 0)),
        out_shape=jax.ShapeDtypeStruct((NP, 128), jnp.float32),
    )(spp, g, dinv, b, wn)


def _fin_body(spp_ref, g_ref, dinv_ref, b4_ref,
              pw1_ref, pb1_ref, pw2_ref, pb2_ref,
              vw1_ref, vb1_ref, vw2_ref, vb2_ref,
              proba_ref, value_ref, accum):
    i = pl.program_id(0)
    dinv = dinv_ref[...]
    sp = spp_ref[...]
    v = (sp[0] + sp[1] + g_ref[...]) * dinv + b4_ref[...]
    h = jnp.where(v > 0, v, jnp.exp(v) - 1.0)
    p = jnp.dot(h, pw1_ref[...], precision=_HI) + pb1_ref[...]
    p = jnp.where(p > 0, p, jnp.exp(p) - 1.0)
    proba_ref[...] = jnp.dot(p, pw2_ref[...], precision=_HI) + pb2_ref[...]

    rid = lax.broadcasted_iota(jnp.int32, (BTP, 1), 0)
    hmask = jnp.where(rid < (N // 8 - i * BTP), h, 0.0)
    bs = jnp.sum(hmask, axis=0, keepdims=True)

    @pl.when(i == 0)
    def _():
        accum[...] = bs

    @pl.when(i > 0)
    def _():
        accum[...] = accum[...] + bs

    @pl.when(i == GRID - 1)
    def _():
        a = accum[...]
        m = (a[:, 0:16] + a[:, 16:32] + a[:, 32:48] + a[:, 48:64]
             + a[:, 64:80] + a[:, 80:96] + a[:, 96:112] + a[:, 112:128])
        m = m * (1.0 / N)
        vv = jnp.dot(m, vw1_ref[...], precision=_HI) + vb1_ref[...]
        vv = jnp.where(vv > 0, vv, jnp.exp(vv) - 1.0)
        value_ref[...] = jnp.dot(vv, vw2_ref[...], precision=_HI) + vb2_ref[...]


def _fin_call(spp, g, dinv, b4, pw1, pb1, pw2, pb2, vw1, vb1, vw2, vb2):
    return pl.pallas_call(
        _fin_body,
        grid=(GRID,),
        in_specs=[
            pl.BlockSpec((2, BTP, 128), lambda i: (0, i, 0)),
            pl.BlockSpec((BTP, 128), lambda i: (i, 0)),
            pl.BlockSpec((BTP, 128), lambda i: (i, 0)),
            pl.BlockSpec((1, 128), lambda i: (0, 0)),
            pl.BlockSpec((128, 128), lambda i: (0, 0)),
            pl.BlockSpec((1, 128), lambda i: (0, 0)),
            pl.BlockSpec((128, 8), lambda i: (0, 0)),
            pl.BlockSpec((1, 1), lambda i: (0, 0)),
            pl.BlockSpec((F, F), lambda i: (0, 0)),
            pl.BlockSpec((1, F), lambda i: (0, 0)),
            pl.BlockSpec((F, 1), lambda i: (0, 0)),
            pl.BlockSpec((1, 1), lambda i: (0, 0)),
        ],
        out_specs=[
            pl.BlockSpec((BTP, 8), lambda i: (i, 0)),
            pl.BlockSpec((1, 1), lambda i: (0, 0)),
        ],
        out_shape=[
            jax.ShapeDtypeStruct((NP, 8), jnp.float32),
            jax.ShapeDtypeStruct((1, 1), jnp.float32),
        ],
        scratch_shapes=[pltpu.VMEM((1, 128), jnp.float32)],
    )(spp, g, dinv, b4, pw1, pb1, pw2, pb2, vw1, vb1, vw2, vb2)


def kernel(x, edge_index, pre_W1, pre_b1, pre_W2, pre_b2, conv_W, conv_b,
           pol_W1, pol_b1, pol_W2, pol_b2, val_W1, val_b1, val_W2, val_b2):
    e = edge_index.shape[1]
    eflat = edge_index.astype(jnp.int32).reshape(-1)
    src = eflat[:e]
    dst = eflat[e:]
    npad = EPAD - e
    ar = jnp.arange(npad, dtype=jnp.int32)
    srcp = jnp.concatenate([src, ar % 256]).reshape(NW, NBW, BS)
    dstp = jnp.concatenate([dst, N + ar % (NPAD - N)]).reshape(NW, NBW, BS)

    degp = _deg_kernel(dstp)
    g1u = _pre_call(x, pre_W1, pre_b1.reshape(1, -1),
                    pre_W2, pre_b2.reshape(1, -1), conv_W[0])
    dinv1 = _dinv_call(degp.reshape(-1, 128))

    eye8 = jnp.eye(8, dtype=jnp.float32)
    zrows = jnp.zeros((NPAD, F), jnp.float32)
    dinvp = jnp.broadcast_to(dinv1.reshape(NPAD, 1), (NPAD, F)).reshape(NP, 128)
    gp = jnp.pad(g1u.reshape(N // 8, 128), ((0, NP - N // 8), (0, 0))) * dinvp

    for i in range(3):
        spp = _scatter_kernel(srcp, dstp, gp.reshape(NPAD, F), zrows)
        gp = _mid_call(spp.reshape(2, NP, 128), gp, dinvp,
                       jnp.tile(conv_b[i], 8).reshape(1, 128),
                       jnp.kron(eye8, conv_W[i + 1]))
    spp = _scatter_kernel(srcp, dstp, gp.reshape(NPAD, F), zrows)
    probap, value = _fin_call(
        spp.reshape(2, NP, 128), gp, dinvp,
        jnp.tile(conv_b[3], 8).reshape(1, 128),
        jnp.kron(eye8, pol_W1), jnp.tile(pol_b1, 8).reshape(1, 128),
        jnp.kron(eye8, pol_W2), pol_b2.reshape(1, 1),
        val_W1, val_b1.reshape(1, -1), val_W2, val_b2.reshape(1, 1))
    proba = probap.reshape(NPAD, 1)[:N]
    return (proba, value)
